# Initial kernel scaffold; baseline (speedup 1.0000x reference)
import jax
import jax.numpy as jnp
from jax.experimental import pallas as pl


def kernel(x, edge_index, edge_attr, receiver_mask, batch, W_msg1, b_msg1, W_upd1, b_upd1, W_msg2, b_msg2, W_upd2, b_upd2, W_recv, b_recv, W_shot1, b_shot1, W_shot2, b_shot2):
    raise NotImplementedError("write your pallas kernel here")


# speedup vs baseline: 2.5797x; 2.5797x over previous
import functools

import jax
import jax.numpy as jnp
from jax import lax
from jax.experimental import pallas as pl
from jax.experimental.pallas import tpu as pltpu
from jax.experimental.pallas import tpu_sc as plsc

N = 50000
E = 800000
B = 1000
H = 64
HH = 32
BN = 1000
NBN = N // BN
BE = 2000
NBE = E // BE
BL = 1024
NEG = -1e30

NSUB = 16
EPT = E // NSUB
C = 400
NCHUNK = EPT // C
GS = 80
NG = C // GS
RZ = N // NSUB
ZR = 625


def _msg_body(p_st, q_st, src_h, dst_h, agg_h, idxb, rowsb, qb, dstb,
              agg_sp, sem):
    c = lax.axis_index("c")
    s = lax.axis_index("s")

    z16 = jnp.zeros((16,), jnp.float32)

    def zb(i, carry):
        rowsb[i, pl.ds(0, 16)] = z16
        rowsb[i, pl.ds(16, 16)] = z16
        return carry

    lax.fori_loop(0, C, zb, 0)
    for k in range(RZ // C):
        pltpu.sync_copy(rowsb, agg_sp.at[pl.ds(s * RZ + k * C, C)])
    pltpu.sync_copy(rowsb.at[pl.ds(0, RZ % C)],
                    agg_sp.at[pl.ds(s * RZ + (RZ // C) * C, RZ % C)])
    plsc.subcore_barrier()

    base0 = s * EPT
    coff = c * N

    def chunk(g, carry):
        base = base0 + g * C
        pltpu.sync_copy(src_h.at[pl.ds(base, C)], idxb)
        for r in range(C // 16):
            idxb[pl.ds(r * 16, 16)] = idxb[pl.ds(r * 16, 16)] + coff
        descs = []
        for j in range(NG):
            descs.append(
                pltpu.async_copy(
                    p_st.at[idxb.at[pl.ds(j * GS, GS)]],
                    rowsb.at[pl.ds(j * GS, GS)], sem))
        pltpu.sync_copy(q_st.at[pl.ds(c * E + base, C)], qb)
        pltpu.sync_copy(dst_h.at[pl.ds(base // GS, NG)], dstb)
        for d in descs:
            d.wait()
        U = 8

        def mb(i, carry):
            for u in range(U):
                e = i * U + u
                a = rowsb[e, pl.ds(0, 16)] + qb[e, pl.ds(0, 16)]
                rowsb[e, pl.ds(0, 16)] = jnp.maximum(a, 0.0)
                b2 = rowsb[e, pl.ds(16, 16)] + qb[e, pl.ds(16, 16)]
                rowsb[e, pl.ds(16, 16)] = jnp.maximum(b2, 0.0)
            return carry

        lax.fori_loop(0, C // U, mb, 0)
        for j in range(NG):
            pltpu.sync_copy(rowsb.at[pl.ds(j * GS, GS)],
                            agg_sp.at[dstb.at[j]], add=True)
        return carry

    lax.fori_loop(0, NCHUNK, chunk, 0)
    plsc.subcore_barrier()
    pltpu.sync_copy(agg_sp.at[pl.ds(s * RZ, RZ)],
                    agg_h.at[pl.ds(coff + s * RZ, RZ)])


@functools.cache
def _get_msg_call():
  return pl.kernel(
    _msg_body,
    out_type=jax.ShapeDtypeStruct((2 * N, HH), jnp.float32),
    mesh=plsc.VectorSubcoreMesh(core_axis_name="c", subcore_axis_name="s"),
    compiler_params=pltpu.CompilerParams(use_tc_tiling_on_sc=False),
    scratch_types=[
        pltpu.VMEM((C,), jnp.int32),
        pltpu.VMEM((C, HH), jnp.float32),
        pltpu.VMEM((C, HH), jnp.float32),
        pltpu.VMEM((NG, GS), jnp.int32),
        pltpu.VMEM_SHARED((N, HH), jnp.float32),
        pltpu.SemaphoreType.DMA,
    ],
  )


def _sel(c, lo, hi):
    return jnp.where(c == 0, lo, hi)


def _pre_k(x, wmlo, wmhi, bmlo, bmhi, wulo, wuhi, bulo, buhi, p_st, xu_st):
    c = pl.program_id(0)
    xb = x[...]
    p_st[...] = jnp.dot(xb, _sel(c, wmlo[...], wmhi[...]),
                        preferred_element_type=jnp.float32) + _sel(
                            c, bmlo[...], bmhi[...])
    xu_st[...] = jnp.dot(xb, _sel(c, wulo[...], wuhi[...]),
                         preferred_element_type=jnp.float32) + _sel(
                             c, bulo[...], buhi[...])


def _qtab_k(attr, w1lo, w1hi, w2lo, w2hi, q1_st, q2_st):
    c = pl.program_id(0)
    ab = attr[...]
    q1_st[...] = jnp.dot(ab, _sel(c, w1lo[...], w1hi[...]),
                         preferred_element_type=jnp.float32)
    q2_st[...] = jnp.dot(ab, _sel(c, w2lo[...], w2hi[...]),
                         preferred_element_type=jnp.float32)


def _upd_k(base_st, agg_lo, agg_hi, wa_lo, wa_hi, wb_lo, wb_hi, h_st):
    c = pl.program_id(0)
    acc = base_st[...]
    acc += jnp.dot(agg_lo[...], _sel(c, wa_lo[...], wa_hi[...]),
                   preferred_element_type=jnp.float32)
    acc += jnp.dot(agg_hi[...], _sel(c, wb_lo[...], wb_hi[...]),
                   preferred_element_type=jnp.float32)
    h_st[...] = jnp.maximum(acc, 0.0)


def _tab2_k(h_lo, h_hi, wa_ll, wa_lh, wa_hl, wa_hh, ba_lo, ba_hi,
            wb_ll, wb_lh, wb_hl, wb_hh, bb_lo, bb_hi, a_st, b_st):
    c = pl.program_id(0)
    hl = h_lo[...]
    hh = h_hi[...]
    a_st[...] = (jnp.dot(hl, _sel(c, wa_ll[...], wa_lh[...]),
                         preferred_element_type=jnp.float32)
                 + jnp.dot(hh, _sel(c, wa_hl[...], wa_hh[...]),
                           preferred_element_type=jnp.float32)
                 + _sel(c, ba_lo[...], ba_hi[...]))
    b_st[...] = (jnp.dot(hl, _sel(c, wb_ll[...], wb_lh[...]),
                         preferred_element_type=jnp.float32)
                 + jnp.dot(hh, _sel(c, wb_hl[...], wb_hh[...]),
                           preferred_element_type=jnp.float32)
                 + _sel(c, bb_lo[...], bb_hi[...]))


def _logits_k(hu_st, agg_lo, agg_hi, wlo, whi, wr_lo, wr_hi, br, logits):
    h2lo = jnp.maximum(
        hu_st[0] + jnp.dot(agg_lo[...], wlo[0],
                           preferred_element_type=jnp.float32)
        + jnp.dot(agg_hi[...], whi[0], preferred_element_type=jnp.float32),
        0.0)
    h2hi = jnp.maximum(
        hu_st[1] + jnp.dot(agg_lo[...], wlo[1],
                           preferred_element_type=jnp.float32)
        + jnp.dot(agg_hi[...], whi[1], preferred_element_type=jnp.float32),
        0.0)
    lg = (jnp.dot(h2lo, wr_lo[...], preferred_element_type=jnp.float32)
          + jnp.dot(h2hi, wr_hi[...], preferred_element_type=jnp.float32)
          + br[...])
    logits[...] = lg


def _segmax_k(logits, maskf, batch, m_out, macc):
    g = pl.program_id(0)

    @pl.when(g == 0)
    def _():
        macc[...] = jnp.full((1, BL), -jnp.inf, jnp.float32)

    oh = batch[...] == lax.broadcasted_iota(jnp.int32, (BN, BL), 1)
    ml = jnp.where(maskf[...] > 0, logits[...], NEG)
    mx = jnp.max(jnp.where(oh, ml, -jnp.inf), axis=0, keepdims=True)
    macc[...] = jnp.maximum(macc[...], mx)

    @pl.when(g == NBN - 1)
    def _():
        mm = macc[...]
        m_out[...] = jnp.where(jnp.isfinite(mm), mm, 0.0)


def _exp_k(logits, maskf, batch, m, e_out, denom, dacc):
    g = pl.program_id(0)

    @pl.when(g == 0)
    def _():
        dacc[...] = jnp.zeros((1, BL), jnp.float32)

    oh = batch[...] == lax.broadcasted_iota(jnp.int32, (BN, BL), 1)
    mg = jnp.sum(jnp.where(oh, m[...], 0.0), axis=1, keepdims=True)
    z = jnp.where(maskf[...] > 0, logits[...] - mg, NEG)
    e = jnp.exp(z)
    e_out[...] = e
    dacc[...] += jnp.sum(jnp.where(oh, e, 0.0), axis=0, keepdims=True)

    @pl.when(g == NBN - 1)
    def _():
        denom[...] = dacc[...]


def _probs_k(e_in, batch, denom, probs, pm_out, ss_out, pmacc, ssacc):
    g = pl.program_id(0)

    @pl.when(g == 0)
    def _():
        pmacc[...] = jnp.full((1, BL), -jnp.inf, jnp.float32)
        ssacc[...] = jnp.zeros((1, BL), jnp.float32)

    oh = batch[...] == lax.broadcasted_iota(jnp.int32, (BN, BL), 1)
    dg = jnp.sum(jnp.where(oh, denom[...], 0.0), axis=1, keepdims=True)
    p = e_in[...] / (dg + 1e-12)
    probs[...] = p
    pmacc[...] = jnp.maximum(
        pmacc[...], jnp.max(jnp.where(oh, p, -jnp.inf), axis=0,
                            keepdims=True))
    ssacc[...] += jnp.sum(jnp.where(oh, p, 0.0), axis=0, keepdims=True)

    @pl.when(g == NBN - 1)
    def _():
        pm_out[...] = pmacc[...]
        ss_out[...] = ssacc[...]


def _first_k(probs, batch, pm, ss, tgt_out, facc):
    g = pl.program_id(0)
    imax = jnp.int32(2147483647)

    @pl.when(g == 0)
    def _():
        facc[...] = jnp.full((1, BL), imax, jnp.int32)

    oh = batch[...] == lax.broadcasted_iota(jnp.int32, (BN, BL), 1)
    pg = jnp.sum(jnp.where(oh, pm[...], 0.0), axis=1, keepdims=True)
    idxv = (g * BN
            + lax.broadcasted_iota(jnp.int32, (BN, 1), 0))
    cand = jnp.where(probs[...] == pg, idxv, jnp.int32(N))
    cmin = jnp.min(jnp.where(oh, cand, imax), axis=0, keepdims=True)
    facc[...] = jnp.minimum(facc[...], cmin)

    @pl.when(g == NBN - 1)
    def _():
        tgt_out[...] = jnp.where((ss[...] > 0) & (facc[...] < N),
                                 facc[...], jnp.int32(N))


def _fix_k(batch, tgt, p1_st, xu_st, wm_lo, wm_hi, wu_lo, wu_hi,
           ind_out, p1p_st, xup_st):
    c = pl.program_id(0)
    g = pl.program_id(1)
    oh = batch[...] == lax.broadcasted_iota(jnp.int32, (BN, BL), 1)
    tg = jnp.sum(jnp.where(oh, tgt[...], 0), axis=1, keepdims=True)
    idxv = g * BN + lax.broadcasted_iota(jnp.int32, (BN, 1), 0)
    ind = jnp.where(idxv == tg, 1.0, 0.0).astype(jnp.float32)
    ind_out[...] = ind
    p1p_st[...] = p1_st[...] + ind * _sel(c, wm_lo[...], wm_hi[...])
    xup_st[...] = xu_st[...] + ind * _sel(c, wu_lo[...], wu_hi[...])


def _pool_k(hu_st, agg_lo, agg_hi, wlo, whi, batch, pooled, counts,
            pacc, cacc):
    c = pl.program_id(0)
    g = pl.program_id(1)

    @pl.when(g == 0)
    def _():
        pacc[...] = jnp.zeros((BL, HH), jnp.float32)
        cacc[...] = jnp.zeros((1, BL), jnp.float32)

    h2 = jnp.maximum(
        hu_st[...] + jnp.dot(agg_lo[...], _sel(c, wlo[0], wlo[1]),
                             preferred_element_type=jnp.float32)
        + jnp.dot(agg_hi[...], _sel(c, whi[0], whi[1]),
                  preferred_element_type=jnp.float32), 0.0)
    ohf = (batch[...] == lax.broadcasted_iota(jnp.int32, (BN, BL), 1)
           ).astype(jnp.float32)
    pacc[...] += lax.dot_general(ohf, h2, (((0,), (0,)), ((), ())),
                                 preferred_element_type=jnp.float32,
                                 precision=lax.Precision.HIGHEST)
    cacc[...] += jnp.sum(ohf, axis=0, keepdims=True)

    @pl.when(g == NBN - 1)
    def _():
        pooled[0] = pacc[...]
        counts[...] = cacc[...]


def _shot_k(pooled, counts, w1_lo, w1_hi, b1, w2, b2, shot):
    cnt = jnp.maximum(counts[...], 1.0)
    inv = (1.0 / cnt).reshape(BL, 1)
    emb_lo = pooled[0] * inv
    emb_hi = pooled[1] * inv
    s = jnp.maximum(
        jnp.dot(emb_lo, w1_lo[...], preferred_element_type=jnp.float32)
        + jnp.dot(emb_hi, w1_hi[...], preferred_element_type=jnp.float32)
        + b1[...], 0.0)
    lg = jnp.dot(s, w2[...], preferred_element_type=jnp.float32) + b2[...]
    shot[...] = lg[:B, :]


def _vspec(shape):
    return pl.BlockSpec(shape, lambda *args: tuple(0 for _ in shape))


def kernel(x, edge_index, edge_attr, receiver_mask, batch,
           W_msg1, b_msg1, W_upd1, b_upd1, W_msg2, b_msg2, W_upd2, b_upd2,
           W_recv, b_recv, W_shot1, b_shot1, W_shot2, b_shot2):
    f32 = jnp.float32
    src = edge_index[0]
    dst = edge_index[1]
    dst2d = dst.reshape(E // GS, GS)
    maskf = receiver_mask.astype(f32).reshape(N, 1)
    batch2 = batch.reshape(N, 1)

    wm1x_lo, wm1x_hi = W_msg1[:13, :HH], W_msg1[:13, HH:]
    _b16 = lambda w: w.astype(jnp.bfloat16).astype(jnp.float32)
    wm1i_lo, wm1i_hi = _b16(W_msg1[13:14, :HH]), _b16(W_msg1[13:14, HH:])
    wm1a_lo, wm1a_hi = W_msg1[14:18, :HH], W_msg1[14:18, HH:]
    bm1_lo, bm1_hi = b_msg1.reshape(1, H)[:, :HH], b_msg1.reshape(1, H)[:, HH:]
    wu1x_lo, wu1x_hi = W_upd1[:13, :HH], W_upd1[:13, HH:]
    wu1i_lo, wu1i_hi = _b16(W_upd1[13:14, :HH]), _b16(W_upd1[13:14, HH:])
    wu1a = W_upd1[14:78]
    wu1a_ll, wu1a_lh = wu1a[:HH, :HH], wu1a[:HH, HH:]
    wu1a_hl, wu1a_hh = wu1a[HH:, :HH], wu1a[HH:, HH:]
    bu1_lo, bu1_hi = b_upd1.reshape(1, H)[:, :HH], b_upd1.reshape(1, H)[:, HH:]
    wm2x = W_msg2[:64]
    wm2a_lo, wm2a_hi = W_msg2[64:68, :HH], W_msg2[64:68, HH:]
    bm2 = b_msg2.reshape(1, H)
    wu2h = W_upd2[:64]
    wu2a = W_upd2[64:128]
    bu2 = b_upd2.reshape(1, H)
    wu2a_lo = jnp.stack([wu2a[:HH, :HH], wu2a[:HH, HH:]])
    wu2a_hi = jnp.stack([wu2a[HH:, :HH], wu2a[HH:, HH:]])
    wr_lo = W_recv[:HH]
    wr_hi = W_recv[HH:]
    br = b_recv.reshape(1, 1)
    ws1_lo, ws1_hi = W_shot1[:HH], W_shot1[HH:]
    bs1 = b_shot1.reshape(1, H)
    ws2 = W_shot2
    bs2 = b_shot2.reshape(1, 1)

    cost_big = pl.CostEstimate(flops=2 * N * 78 * H, bytes_accessed=N * 600,
                               transcendentals=0)

    wspec13 = _vspec((13, HH))
    bspec = _vspec((1, HH))
    p1_st, xu_st = pl.pallas_call(
        _pre_k,
        grid=(2, NBN),
        in_specs=[pl.BlockSpec((BN, 13), lambda c, g: (g, 0))] +
                 [wspec13, wspec13, bspec, bspec, wspec13, wspec13, bspec,
                  bspec],
        out_specs=[pl.BlockSpec((BN, HH), lambda c, g: (c * NBN + g, 0))] * 2,
        out_shape=[jax.ShapeDtypeStruct((2 * N, HH), f32)] * 2,
    )(x, wm1x_lo, wm1x_hi, bm1_lo, bm1_hi, wu1x_lo, wu1x_hi, bu1_lo, bu1_hi)

    wspec4 = _vspec((4, HH))
    q1_st, q2_st = pl.pallas_call(
        _qtab_k,
        grid=(2, NBE),
        in_specs=[pl.BlockSpec((BE, 4), lambda c, g: (g, 0)),
                  wspec4, wspec4, wspec4, wspec4],
        out_specs=[pl.BlockSpec((BE, HH), lambda c, g: (c * NBE + g, 0))] * 2,
        out_shape=[jax.ShapeDtypeStruct((2 * E, HH), f32)] * 2,
    )(edge_attr, wm1a_lo, wm1a_hi, wm2a_lo, wm2a_hi)

    wspecH = _vspec((HH, HH))

    def upd(base_st, agg_st, wll, wlh, whl, whh):
        return pl.pallas_call(
            _upd_k,
            grid=(2, NBN),
            in_specs=[pl.BlockSpec((BN, HH), lambda c, g: (c * NBN + g, 0)),
                      pl.BlockSpec((BN, HH), lambda c, g: (g, 0)),
                      pl.BlockSpec((BN, HH), lambda c, g: (NBN + g, 0)),
                      wspecH, wspecH, wspecH, wspecH],
            out_specs=pl.BlockSpec((BN, HH), lambda c, g: (c * NBN + g, 0)),
            out_shape=jax.ShapeDtypeStruct((2 * N, HH), f32),
            cost_estimate=cost_big,
        )(base_st, agg_st, agg_st, wll, wlh, whl, whh)

    def tab2(h_st, WA, bA, WB, bB):
        return pl.pallas_call(
            _tab2_k,
            grid=(2, NBN),
            in_specs=[pl.BlockSpec((BN, HH), lambda c, g: (g, 0)),
                      pl.BlockSpec((BN, HH), lambda c, g: (NBN + g, 0)),
                      wspecH, wspecH, wspecH, wspecH, bspec, bspec,
                      wspecH, wspecH, wspecH, wspecH, bspec, bspec],
            out_specs=[pl.BlockSpec((BN, HH),
                                    lambda c, g: (c * NBN + g, 0))] * 2,
            out_shape=[jax.ShapeDtypeStruct((2 * N, HH), f32)] * 2,
            cost_estimate=cost_big,
        )(h_st, h_st,
          WA[:HH, :HH], WA[:HH, HH:], WA[HH:, :HH], WA[HH:, HH:],
          bA[:, :HH], bA[:, HH:],
          WB[:HH, :HH], WB[:HH, HH:], WB[HH:, :HH], WB[HH:, HH:],
          bB[:, :HH], bB[:, HH:])

    def backbone_tail(p1t, xut):
        msg = _get_msg_call()
        agg1_st = msg(p1t, q1_st, src, dst2d)
        h1_st = upd(xut, agg1_st, wu1a_ll, wu1a_lh, wu1a_hl, wu1a_hh)
        p2_st, hu_st = tab2(h1_st, wm2x, bm2, wu2h, bu2)
        agg2_st = msg(p2_st, q2_st, src, dst2d)
        return hu_st, agg2_st

    hu_st, agg2_st = backbone_tail(p1_st, xu_st)
    logits = pl.pallas_call(
        _logits_k,
        grid=(NBN,),
        in_specs=[pl.BlockSpec((2, BN, HH), lambda g: (0, g, 0)),
                  pl.BlockSpec((BN, HH), lambda g: (g, 0)),
                  pl.BlockSpec((BN, HH), lambda g: (NBN + g, 0)),
                  _vspec((2, HH, HH)), _vspec((2, HH, HH)),
                  _vspec((HH, 1)), _vspec((HH, 1)), _vspec((1, 1))],
        out_specs=pl.BlockSpec((BN, 1), lambda g: (g, 0)),
        out_shape=jax.ShapeDtypeStruct((N, 1), f32),
        cost_estimate=cost_big,
    )(hu_st.reshape(2, N, HH), agg2_st, agg2_st, wu2a_lo, wu2a_hi,
      wr_lo, wr_hi, br)

    nspec = pl.BlockSpec((BN, 1), lambda g: (g, 0))
    bl_spec = _vspec((1, BL))
    segargs = dict(grid=(NBN,))
    m_seg = pl.pallas_call(
        _segmax_k, in_specs=[nspec, nspec, nspec], out_specs=bl_spec,
        out_shape=jax.ShapeDtypeStruct((1, BL), f32),
        scratch_shapes=[pltpu.VMEM((1, BL), f32)], **segargs,
    )(logits, maskf, batch2)
    e_arr, denom = pl.pallas_call(
        _exp_k, in_specs=[nspec, nspec, nspec, bl_spec],
        out_specs=[nspec, bl_spec],
        out_shape=[jax.ShapeDtypeStruct((N, 1), f32),
                   jax.ShapeDtypeStruct((1, BL), f32)],
        scratch_shapes=[pltpu.VMEM((1, BL), f32)], **segargs,
    )(logits, maskf, batch2, m_seg)
    probs, pm, ss = pl.pallas_call(
        _probs_k, in_specs=[nspec, nspec, bl_spec],
        out_specs=[nspec, bl_spec, bl_spec],
        out_shape=[jax.ShapeDtypeStruct((N, 1), f32),
                   jax.ShapeDtypeStruct((1, BL), f32),
                   jax.ShapeDtypeStruct((1, BL), f32)],
        scratch_shapes=[pltpu.VMEM((1, BL), f32)] * 2, **segargs,
    )(e_arr, batch2, denom)
    tgt = pl.pallas_call(
        _first_k, in_specs=[nspec, nspec, bl_spec, bl_spec],
        out_specs=pl.BlockSpec((1, BL), lambda g: (0, 0)),
        out_shape=jax.ShapeDtypeStruct((1, BL), jnp.int32),
        scratch_shapes=[pltpu.VMEM((1, BL), jnp.int32)], **segargs,
    )(probs, batch2, pm, ss)

    wspec1 = _vspec((1, HH))
    ind, p1p_st, xup_st = pl.pallas_call(
        _fix_k,
        grid=(2, NBN),
        in_specs=[pl.BlockSpec((BN, 1), lambda c, g: (g, 0)),
                  pl.BlockSpec((1, BL), lambda c, g: (0, 0)),
                  pl.BlockSpec((BN, HH), lambda c, g: (c * NBN + g, 0)),
                  pl.BlockSpec((BN, HH), lambda c, g: (c * NBN + g, 0)),
                  wspec1, wspec1, wspec1, wspec1],
        out_specs=[pl.BlockSpec((BN, 1), lambda c, g: (g, 0)),
                   pl.BlockSpec((BN, HH), lambda c, g: (c * NBN + g, 0)),
                   pl.BlockSpec((BN, HH), lambda c, g: (c * NBN + g, 0))],
        out_shape=[jax.ShapeDtypeStruct((N, 1), f32),
                   jax.ShapeDtypeStruct((2 * N, HH), f32),
                   jax.ShapeDtypeStruct((2 * N, HH), f32)],
    )(batch2, tgt, p1_st, xu_st, wm1i_lo, wm1i_hi, wu1i_lo, wu1i_hi)

    hu2_st, agg2b_st = backbone_tail(p1p_st, xup_st)
    pooled, counts = pl.pallas_call(
        _pool_k,
        grid=(2, NBN),
        in_specs=[pl.BlockSpec((BN, HH), lambda c, g: (c * NBN + g, 0)),
                  pl.BlockSpec((BN, HH), lambda c, g: (g, 0)),
                  pl.BlockSpec((BN, HH), lambda c, g: (NBN + g, 0)),
                  _vspec((2, HH, HH)), _vspec((2, HH, HH)),
                  pl.BlockSpec((BN, 1), lambda c, g: (g, 0))],
        out_specs=[pl.BlockSpec((1, BL, HH), lambda c, g: (c, 0, 0)),
                   pl.BlockSpec((1, BL), lambda c, g: (0, 0))],
        out_shape=[jax.ShapeDtypeStruct((2, BL, HH), f32),
                   jax.ShapeDtypeStruct((1, BL), f32)],
        scratch_shapes=[pltpu.VMEM((BL, HH), f32), pltpu.VMEM((1, BL), f32)],
        cost_estimate=cost_big,
    )(hu2_st, agg2b_st, agg2b_st, wu2a_lo, wu2a_hi, batch2)

    shot = pl.pallas_call(
        _shot_k,
        in_specs=[_vspec((2, BL, HH)), _vspec((1, BL)),
                  _vspec((HH, H)), _vspec((HH, H)), _vspec((1, H)),
                  _vspec((H, 1)), _vspec((1, 1))],
        out_specs=_vspec((B, 1)),
        out_shape=jax.ShapeDtypeStruct((B, 1), f32),
    )(pooled, counts, ws1_lo, ws1_hi, bs1, ws2, bs2)

    return probs[:, 0], ind[:, 0], shot


# speedup vs baseline: 2.8333x; 1.0983x over previous
import functools

import jax
import jax.numpy as jnp
from jax import lax
from jax.experimental import pallas as pl
from jax.experimental.pallas import tpu as pltpu
from jax.experimental.pallas import tpu_sc as plsc

N = 50000
E = 800000
B = 1000
H = 64
HH = 32
BN = 1000
NBN = N // BN
BE = 2000
NBE = E // BE
BL = 1024
NEG = -1e30

NSUB = 16
EPT = E // NSUB
C = 200
NCHUNK = EPT // C
NPAIR = NCHUNK // 2
GS = 40
NG = C // GS
CPAD = 208
RZ = N // NSUB


def _msg_body(p_st, q_st, src_h, dst_h, agg_h,
              idx0, idx1, rows0, rows1, q0, q1, d0, d1,
              agg_sp, sin0, sin1, sg0, sg1):
    c = lax.axis_index("c")
    s = lax.axis_index("s")
    slots = ((idx0, rows0, q0, d0, sin0, sg0),
             (idx1, rows1, q1, d1, sin1, sg1))

    z16 = jnp.zeros((16,), jnp.float32)

    def zb(i, carry):
        rows0[i, pl.ds(0, 16)] = z16
        rows0[i, pl.ds(16, 16)] = z16
        return carry

    lax.fori_loop(0, C, zb, 0)
    for k in range(RZ // C):
        pltpu.sync_copy(rows0, agg_sp.at[pl.ds(s * RZ + k * C, C)])
    pltpu.sync_copy(rows0.at[pl.ds(0, RZ % C)],
                    agg_sp.at[pl.ds(s * RZ + (RZ // C) * C, RZ % C)])
    plsc.subcore_barrier()

    base0 = s * EPT
    coff = c * N

    def in_copies(ch, sl):
        idxb, rowsb, qb, dstb, sin, sg = sl
        base = base0 + ch * C
        return (
            pltpu.make_async_copy(src_h.at[pl.ds(base, C)],
                                  idxb.at[pl.ds(0, C)], sin),
            pltpu.make_async_copy(q_st.at[pl.ds(c * E + base, C)], qb, sin),
            pltpu.make_async_copy(dst_h.at[pl.ds(base // GS, NG)], dstb, sin),
        )

    def issue_in(ch, sl):
        for cp in in_copies(ch, sl):
            cp.start()

    def wait_in(ch, sl):
        for cp in in_copies(ch, sl):
            cp.wait()

    def gather_copies(sl):
        idxb, rowsb, qb, dstb, sin, sg = sl
        return tuple(
            pltpu.make_async_copy(p_st.at[idxb.at[pl.ds(j * GS, GS)]],
                                  rowsb.at[pl.ds(j * GS, GS)], sg)
            for j in range(NG))

    def idx_add_and_gather(sl):
        idxb = sl[0]
        for r in range(CPAD // 16):
            idxb[pl.ds(r * 16, 16)] = idxb[pl.ds(r * 16, 16)] + coff
        for cp in gather_copies(sl):
            cp.start()

    def compute_scatter(sl):
        idxb, rowsb, qb, dstb, sin, sg = sl
        for cp in gather_copies(sl):
            cp.wait()
        U = 8

        def mb(i, carry):
            for u in range(U):
                e = i * U + u
                a = rowsb[e, pl.ds(0, 16)] + qb[e, pl.ds(0, 16)]
                rowsb[e, pl.ds(0, 16)] = jnp.maximum(a, 0.0)
                b2 = rowsb[e, pl.ds(16, 16)] + qb[e, pl.ds(16, 16)]
                rowsb[e, pl.ds(16, 16)] = jnp.maximum(b2, 0.0)
            return carry

        lax.fori_loop(0, C // U, mb, 0)
        for j in range(NG):
            pltpu.sync_copy(rowsb.at[pl.ds(j * GS, GS)],
                            agg_sp.at[dstb.at[j]], add=True)

    issue_in(0, slots[0])

    def pair(g, carry):
        c0 = 2 * g
        issue_in(c0 + 1, slots[1])
        wait_in(c0, slots[0])
        idx_add_and_gather(slots[0])
        wait_in(c0 + 1, slots[1])
        idx_add_and_gather(slots[1])
        compute_scatter(slots[0])

        @pl.when(g < NPAIR - 1)
        def _():
            issue_in(c0 + 2, slots[0])

        compute_scatter(slots[1])
        return carry

    lax.fori_loop(0, NPAIR, pair, 0)
    plsc.subcore_barrier()
    pltpu.sync_copy(agg_sp.at[pl.ds(s * RZ, RZ)],
                    agg_h.at[pl.ds(coff + s * RZ, RZ)])


@functools.cache
def _get_msg_call():
  return pl.kernel(
    _msg_body,
    out_type=jax.ShapeDtypeStruct((2 * N, HH), jnp.float32),
    mesh=plsc.VectorSubcoreMesh(core_axis_name="c", subcore_axis_name="s"),
    compiler_params=pltpu.CompilerParams(use_tc_tiling_on_sc=False),
    scratch_types=[
        pltpu.VMEM((CPAD,), jnp.int32),
        pltpu.VMEM((CPAD,), jnp.int32),
        pltpu.VMEM((C, HH), jnp.float32),
        pltpu.VMEM((C, HH), jnp.float32),
        pltpu.VMEM((C, HH), jnp.float32),
        pltpu.VMEM((C, HH), jnp.float32),
        pltpu.VMEM((NG, GS), jnp.int32),
        pltpu.VMEM((NG, GS), jnp.int32),
        pltpu.VMEM_SHARED((N, HH), jnp.float32),
        pltpu.SemaphoreType.DMA,
        pltpu.SemaphoreType.DMA,
        pltpu.SemaphoreType.DMA,
        pltpu.SemaphoreType.DMA,
    ],
  )


def _sel(c, lo, hi):
    return jnp.where(c == 0, lo, hi)


def _pre_k(x, wmlo, wmhi, bmlo, bmhi, wulo, wuhi, bulo, buhi, p_st, xu_st):
    c = pl.program_id(0)
    xb = x[...]
    p_st[...] = jnp.dot(xb, _sel(c, wmlo[...], wmhi[...]),
                        preferred_element_type=jnp.float32) + _sel(
                            c, bmlo[...], bmhi[...])
    xu_st[...] = jnp.dot(xb, _sel(c, wulo[...], wuhi[...]),
                         preferred_element_type=jnp.float32) + _sel(
                             c, bulo[...], buhi[...])


def _qtab_k(attr, w1lo, w1hi, w2lo, w2hi, q1_st, q2_st):
    c = pl.program_id(0)
    ab = attr[...]
    q1_st[...] = jnp.dot(ab, _sel(c, w1lo[...], w1hi[...]),
                         preferred_element_type=jnp.float32)
    q2_st[...] = jnp.dot(ab, _sel(c, w2lo[...], w2hi[...]),
                         preferred_element_type=jnp.float32)


def _upd_k(base_st, agg_lo, agg_hi, wa_lo, wa_hi, wb_lo, wb_hi, h_st):
    c = pl.program_id(0)
    acc = base_st[...]
    acc += jnp.dot(agg_lo[...], _sel(c, wa_lo[...], wa_hi[...]),
                   preferred_element_type=jnp.float32)
    acc += jnp.dot(agg_hi[...], _sel(c, wb_lo[...], wb_hi[...]),
                   preferred_element_type=jnp.float32)
    h_st[...] = jnp.maximum(acc, 0.0)


def _tab2_k(h_lo, h_hi, wa_ll, wa_lh, wa_hl, wa_hh, ba_lo, ba_hi,
            wb_ll, wb_lh, wb_hl, wb_hh, bb_lo, bb_hi, a_st, b_st):
    c = pl.program_id(0)
    hl = h_lo[...]
    hh = h_hi[...]
    a_st[...] = (jnp.dot(hl, _sel(c, wa_ll[...], wa_lh[...]),
                         preferred_element_type=jnp.float32)
                 + jnp.dot(hh, _sel(c, wa_hl[...], wa_hh[...]),
                           preferred_element_type=jnp.float32)
                 + _sel(c, ba_lo[...], ba_hi[...]))
    b_st[...] = (jnp.dot(hl, _sel(c, wb_ll[...], wb_lh[...]),
                         preferred_element_type=jnp.float32)
                 + jnp.dot(hh, _sel(c, wb_hl[...], wb_hh[...]),
                           preferred_element_type=jnp.float32)
                 + _sel(c, bb_lo[...], bb_hi[...]))


def _logits_k(hu_st, agg_lo, agg_hi, wlo, whi, wr_lo, wr_hi, br, logits):
    h2lo = jnp.maximum(
        hu_st[0] + jnp.dot(agg_lo[...], wlo[0],
                           preferred_element_type=jnp.float32)
        + jnp.dot(agg_hi[...], whi[0], preferred_element_type=jnp.float32),
        0.0)
    h2hi = jnp.maximum(
        hu_st[1] + jnp.dot(agg_lo[...], wlo[1],
                           preferred_element_type=jnp.float32)
        + jnp.dot(agg_hi[...], whi[1], preferred_element_type=jnp.float32),
        0.0)
    lg = (jnp.dot(h2lo, wr_lo[...], preferred_element_type=jnp.float32)
          + jnp.dot(h2hi, wr_hi[...], preferred_element_type=jnp.float32)
          + br[...])
    logits[...] = lg


def _segmax_k(logits, maskf, batch, m_out, macc):
    g = pl.program_id(0)

    @pl.when(g == 0)
    def _():
        macc[...] = jnp.full((1, BL), -jnp.inf, jnp.float32)

    oh = batch[...] == lax.broadcasted_iota(jnp.int32, (BN, BL), 1)
    ml = jnp.where(maskf[...] > 0, logits[...], NEG)
    mx = jnp.max(jnp.where(oh, ml, -jnp.inf), axis=0, keepdims=True)
    macc[...] = jnp.maximum(macc[...], mx)

    @pl.when(g == NBN - 1)
    def _():
        mm = macc[...]
        m_out[...] = jnp.where(jnp.isfinite(mm), mm, 0.0)


def _exp_k(logits, maskf, batch, m, e_out, denom, dacc):
    g = pl.program_id(0)

    @pl.when(g == 0)
    def _():
        dacc[...] = jnp.zeros((1, BL), jnp.float32)

    oh = batch[...] == lax.broadcasted_iota(jnp.int32, (BN, BL), 1)
    mg = jnp.sum(jnp.where(oh, m[...], 0.0), axis=1, keepdims=True)
    z = jnp.where(maskf[...] > 0, logits[...] - mg, NEG)
    e = jnp.exp(z)
    e_out[...] = e
    dacc[...] += jnp.sum(jnp.where(oh, e, 0.0), axis=0, keepdims=True)

    @pl.when(g == NBN - 1)
    def _():
        denom[...] = dacc[...]


def _probs_k(e_in, batch, denom, probs, pm_out, ss_out, pmacc, ssacc):
    g = pl.program_id(0)

    @pl.when(g == 0)
    def _():
        pmacc[...] = jnp.full((1, BL), -jnp.inf, jnp.float32)
        ssacc[...] = jnp.zeros((1, BL), jnp.float32)

    oh = batch[...] == lax.broadcasted_iota(jnp.int32, (BN, BL), 1)
    dg = jnp.sum(jnp.where(oh, denom[...], 0.0), axis=1, keepdims=True)
    p = e_in[...] / (dg + 1e-12)
    probs[...] = p
    pmacc[...] = jnp.maximum(
        pmacc[...], jnp.max(jnp.where(oh, p, -jnp.inf), axis=0,
                            keepdims=True))
    ssacc[...] += jnp.sum(jnp.where(oh, p, 0.0), axis=0, keepdims=True)

    @pl.when(g == NBN - 1)
    def _():
        pm_out[...] = pmacc[...]
        ss_out[...] = ssacc[...]


def _first_k(probs, batch, pm, ss, tgt_out, facc):
    g = pl.program_id(0)
    imax = jnp.int32(2147483647)

    @pl.when(g == 0)
    def _():
        facc[...] = jnp.full((1, BL), imax, jnp.int32)

    oh = batch[...] == lax.broadcasted_iota(jnp.int32, (BN, BL), 1)
    pg = jnp.sum(jnp.where(oh, pm[...], 0.0), axis=1, keepdims=True)
    idxv = (g * BN
            + lax.broadcasted_iota(jnp.int32, (BN, 1), 0))
    cand = jnp.where(probs[...] == pg, idxv, jnp.int32(N))
    cmin = jnp.min(jnp.where(oh, cand, imax), axis=0, keepdims=True)
    facc[...] = jnp.minimum(facc[...], cmin)

    @pl.when(g == NBN - 1)
    def _():
        tgt_out[...] = jnp.where((ss[...] > 0) & (facc[...] < N),
                                 facc[...], jnp.int32(N))


def _fix_k(batch, tgt, p1_st, xu_st, wm_lo, wm_hi, wu_lo, wu_hi,
           ind_out, p1p_st, xup_st):
    c = pl.program_id(0)
    g = pl.program_id(1)
    oh = batch[...] == lax.broadcasted_iota(jnp.int32, (BN, BL), 1)
    tg = jnp.sum(jnp.where(oh, tgt[...], 0), axis=1, keepdims=True)
    idxv = g * BN + lax.broadcasted_iota(jnp.int32, (BN, 1), 0)
    ind = jnp.where(idxv == tg, 1.0, 0.0).astype(jnp.float32)
    ind_out[...] = ind
    p1p_st[...] = p1_st[...] + ind * _sel(c, wm_lo[...], wm_hi[...])
    xup_st[...] = xu_st[...] + ind * _sel(c, wu_lo[...], wu_hi[...])


def _pool_k(hu_st, agg_lo, agg_hi, wlo, whi, batch, pooled, counts,
            pacc, cacc):
    c = pl.program_id(0)
    g = pl.program_id(1)

    @pl.when(g == 0)
    def _():
        pacc[...] = jnp.zeros((BL, HH), jnp.float32)
        cacc[...] = jnp.zeros((1, BL), jnp.float32)

    h2 = jnp.maximum(
        hu_st[...] + jnp.dot(agg_lo[...], _sel(c, wlo[0], wlo[1]),
                             preferred_element_type=jnp.float32)
        + jnp.dot(agg_hi[...], _sel(c, whi[0], whi[1]),
                  preferred_element_type=jnp.float32), 0.0)
    ohf = (batch[...] == lax.broadcasted_iota(jnp.int32, (BN, BL), 1)
           ).astype(jnp.float32)
    pacc[...] += lax.dot_general(ohf, h2, (((0,), (0,)), ((), ())),
                                 preferred_element_type=jnp.float32,
                                 precision=lax.Precision.HIGHEST)
    cacc[...] += jnp.sum(ohf, axis=0, keepdims=True)

    @pl.when(g == NBN - 1)
    def _():
        pooled[0] = pacc[...]
        counts[...] = cacc[...]


def _shot_k(pooled, counts, w1_lo, w1_hi, b1, w2, b2, shot):
    cnt = jnp.maximum(counts[...], 1.0)
    inv = (1.0 / cnt).reshape(BL, 1)
    emb_lo = pooled[0] * inv
    emb_hi = pooled[1] * inv
    s = jnp.maximum(
        jnp.dot(emb_lo, w1_lo[...], preferred_element_type=jnp.float32)
        + jnp.dot(emb_hi, w1_hi[...], preferred_element_type=jnp.float32)
        + b1[...], 0.0)
    lg = jnp.dot(s, w2[...], preferred_element_type=jnp.float32) + b2[...]
    shot[...] = lg[:B, :]


def _vspec(shape):
    return pl.BlockSpec(shape, lambda *args: tuple(0 for _ in shape))


def kernel(x, edge_index, edge_attr, receiver_mask, batch,
           W_msg1, b_msg1, W_upd1, b_upd1, W_msg2, b_msg2, W_upd2, b_upd2,
           W_recv, b_recv, W_shot1, b_shot1, W_shot2, b_shot2):
    f32 = jnp.float32
    src = edge_index[0]
    dst = edge_index[1]
    dst2d = dst.reshape(E // GS, GS)
    maskf = receiver_mask.astype(f32).reshape(N, 1)
    batch2 = batch.reshape(N, 1)

    wm1x_lo, wm1x_hi = W_msg1[:13, :HH], W_msg1[:13, HH:]
    _b16 = lambda w: w.astype(jnp.bfloat16).astype(jnp.float32)
    wm1i_lo, wm1i_hi = _b16(W_msg1[13:14, :HH]), _b16(W_msg1[13:14, HH:])
    wm1a_lo, wm1a_hi = W_msg1[14:18, :HH], W_msg1[14:18, HH:]
    bm1_lo, bm1_hi = b_msg1.reshape(1, H)[:, :HH], b_msg1.reshape(1, H)[:, HH:]
    wu1x_lo, wu1x_hi = W_upd1[:13, :HH], W_upd1[:13, HH:]
    wu1i_lo, wu1i_hi = _b16(W_upd1[13:14, :HH]), _b16(W_upd1[13:14, HH:])
    wu1a = W_upd1[14:78]
    wu1a_ll, wu1a_lh = wu1a[:HH, :HH], wu1a[:HH, HH:]
    wu1a_hl, wu1a_hh = wu1a[HH:, :HH], wu1a[HH:, HH:]
    bu1_lo, bu1_hi = b_upd1.reshape(1, H)[:, :HH], b_upd1.reshape(1, H)[:, HH:]
    wm2x = W_msg2[:64]
    wm2a_lo, wm2a_hi = W_msg2[64:68, :HH], W_msg2[64:68, HH:]
    bm2 = b_msg2.reshape(1, H)
    wu2h = W_upd2[:64]
    wu2a = W_upd2[64:128]
    bu2 = b_upd2.reshape(1, H)
    wu2a_lo = jnp.stack([wu2a[:HH, :HH], wu2a[:HH, HH:]])
    wu2a_hi = jnp.stack([wu2a[HH:, :HH], wu2a[HH:, HH:]])
    wr_lo = W_recv[:HH]
    wr_hi = W_recv[HH:]
    br = b_recv.reshape(1, 1)
    ws1_lo, ws1_hi = W_shot1[:HH], W_shot1[HH:]
    bs1 = b_shot1.reshape(1, H)
    ws2 = W_shot2
    bs2 = b_shot2.reshape(1, 1)

    cost_big = pl.CostEstimate(flops=2 * N * 78 * H, bytes_accessed=N * 600,
                               transcendentals=0)

    wspec13 = _vspec((13, HH))
    bspec = _vspec((1, HH))
    p1_st, xu_st = pl.pallas_call(
        _pre_k,
        grid=(2, NBN),
        in_specs=[pl.BlockSpec((BN, 13), lambda c, g: (g, 0))] +
                 [wspec13, wspec13, bspec, bspec, wspec13, wspec13, bspec,
                  bspec],
        out_specs=[pl.BlockSpec((BN, HH), lambda c, g: (c * NBN + g, 0))] * 2,
        out_shape=[jax.ShapeDtypeStruct((2 * N, HH), f32)] * 2,
    )(x, wm1x_lo, wm1x_hi, bm1_lo, bm1_hi, wu1x_lo, wu1x_hi, bu1_lo, bu1_hi)

    wspec4 = _vspec((4, HH))
    q1_st, q2_st = pl.pallas_call(
        _qtab_k,
        grid=(2, NBE),
        in_specs=[pl.BlockSpec((BE, 4), lambda c, g: (g, 0)),
                  wspec4, wspec4, wspec4, wspec4],
        out_specs=[pl.BlockSpec((BE, HH), lambda c, g: (c * NBE + g, 0))] * 2,
        out_shape=[jax.ShapeDtypeStruct((2 * E, HH), f32)] * 2,
    )(edge_attr, wm1a_lo, wm1a_hi, wm2a_lo, wm2a_hi)

    wspecH = _vspec((HH, HH))

    def upd(base_st, agg_st, wll, wlh, whl, whh):
        return pl.pallas_call(
            _upd_k,
            grid=(2, NBN),
            in_specs=[pl.BlockSpec((BN, HH), lambda c, g: (c * NBN + g, 0)),
                      pl.BlockSpec((BN, HH), lambda c, g: (g, 0)),
                      pl.BlockSpec((BN, HH), lambda c, g: (NBN + g, 0)),
                      wspecH, wspecH, wspecH, wspecH],
            out_specs=pl.BlockSpec((BN, HH), lambda c, g: (c * NBN + g, 0)),
            out_shape=jax.ShapeDtypeStruct((2 * N, HH), f32),
            cost_estimate=cost_big,
        )(base_st, agg_st, agg_st, wll, wlh, whl, whh)

    def tab2(h_st, WA, bA, WB, bB):
        return pl.pallas_call(
            _tab2_k,
            grid=(2, NBN),
            in_specs=[pl.BlockSpec((BN, HH), lambda c, g: (g, 0)),
                      pl.BlockSpec((BN, HH), lambda c, g: (NBN + g, 0)),
                      wspecH, wspecH, wspecH, wspecH, bspec, bspec,
                      wspecH, wspecH, wspecH, wspecH, bspec, bspec],
            out_specs=[pl.BlockSpec((BN, HH),
                                    lambda c, g: (c * NBN + g, 0))] * 2,
            out_shape=[jax.ShapeDtypeStruct((2 * N, HH), f32)] * 2,
            cost_estimate=cost_big,
        )(h_st, h_st,
          WA[:HH, :HH], WA[:HH, HH:], WA[HH:, :HH], WA[HH:, HH:],
          bA[:, :HH], bA[:, HH:],
          WB[:HH, :HH], WB[:HH, HH:], WB[HH:, :HH], WB[HH:, HH:],
          bB[:, :HH], bB[:, HH:])

    def backbone_tail(p1t, xut):
        msg = _get_msg_call()
        agg1_st = msg(p1t, q1_st, src, dst2d)
        h1_st = upd(xut, agg1_st, wu1a_ll, wu1a_lh, wu1a_hl, wu1a_hh)
        p2_st, hu_st = tab2(h1_st, wm2x, bm2, wu2h, bu2)
        agg2_st = msg(p2_st, q2_st, src, dst2d)
        return hu_st, agg2_st

    hu_st, agg2_st = backbone_tail(p1_st, xu_st)
    logits = pl.pallas_call(
        _logits_k,
        grid=(NBN,),
        in_specs=[pl.BlockSpec((2, BN, HH), lambda g: (0, g, 0)),
                  pl.BlockSpec((BN, HH), lambda g: (g, 0)),
                  pl.BlockSpec((BN, HH), lambda g: (NBN + g, 0)),
                  _vspec((2, HH, HH)), _vspec((2, HH, HH)),
                  _vspec((HH, 1)), _vspec((HH, 1)), _vspec((1, 1))],
        out_specs=pl.BlockSpec((BN, 1), lambda g: (g, 0)),
        out_shape=jax.ShapeDtypeStruct((N, 1), f32),
        cost_estimate=cost_big,
    )(hu_st.reshape(2, N, HH), agg2_st, agg2_st, wu2a_lo, wu2a_hi,
      wr_lo, wr_hi, br)

    nspec = pl.BlockSpec((BN, 1), lambda g: (g, 0))
    bl_spec = _vspec((1, BL))
    segargs = dict(grid=(NBN,))
    m_seg = pl.pallas_call(
        _segmax_k, in_specs=[nspec, nspec, nspec], out_specs=bl_spec,
        out_shape=jax.ShapeDtypeStruct((1, BL), f32),
        scratch_shapes=[pltpu.VMEM((1, BL), f32)], **segargs,
    )(logits, maskf, batch2)
    e_arr, denom = pl.pallas_call(
        _exp_k, in_specs=[nspec, nspec, nspec, bl_spec],
        out_specs=[nspec, bl_spec],
        out_shape=[jax.ShapeDtypeStruct((N, 1), f32),
                   jax.ShapeDtypeStruct((1, BL), f32)],
        scratch_shapes=[pltpu.VMEM((1, BL), f32)], **segargs,
    )(logits, maskf, batch2, m_seg)
    probs, pm, ss = pl.pallas_call(
        _probs_k, in_specs=[nspec, nspec, bl_spec],
        out_specs=[nspec, bl_spec, bl_spec],
        out_shape=[jax.ShapeDtypeStruct((N, 1), f32),
                   jax.ShapeDtypeStruct((1, BL), f32),
                   jax.ShapeDtypeStruct((1, BL), f32)],
        scratch_shapes=[pltpu.VMEM((1, BL), f32)] * 2, **segargs,
    )(e_arr, batch2, denom)
    tgt = pl.pallas_call(
        _first_k, in_specs=[nspec, nspec, bl_spec, bl_spec],
        out_specs=pl.BlockSpec((1, BL), lambda g: (0, 0)),
        out_shape=jax.ShapeDtypeStruct((1, BL), jnp.int32),
        scratch_shapes=[pltpu.VMEM((1, BL), jnp.int32)], **segargs,
    )(probs, batch2, pm, ss)

    wspec1 = _vspec((1, HH))
    ind, p1p_st, xup_st = pl.pallas_call(
        _fix_k,
        grid=(2, NBN),
        in_specs=[pl.BlockSpec((BN, 1), lambda c, g: (g, 0)),
                  pl.BlockSpec((1, BL), lambda c, g: (0, 0)),
                  pl.BlockSpec((BN, HH), lambda c, g: (c * NBN + g, 0)),
                  pl.BlockSpec((BN, HH), lambda c, g: (c * NBN + g, 0)),
                  wspec1, wspec1, wspec1, wspec1],
        out_specs=[pl.BlockSpec((BN, 1), lambda c, g: (g, 0)),
                   pl.BlockSpec((BN, HH), lambda c, g: (c * NBN + g, 0)),
                   pl.BlockSpec((BN, HH), lambda c, g: (c * NBN + g, 0))],
        out_shape=[jax.ShapeDtypeStruct((N, 1), f32),
                   jax.ShapeDtypeStruct((2 * N, HH), f32),
                   jax.ShapeDtypeStruct((2 * N, HH), f32)],
    )(batch2, tgt, p1_st, xu_st, wm1i_lo, wm1i_hi, wu1i_lo, wu1i_hi)

    hu2_st, agg2b_st = backbone_tail(p1p_st, xup_st)
    pooled, counts = pl.pallas_call(
        _pool_k,
        grid=(2, NBN),
        in_specs=[pl.BlockSpec((BN, HH), lambda c, g: (c * NBN + g, 0)),
                  pl.BlockSpec((BN, HH), lambda c, g: (g, 0)),
                  pl.BlockSpec((BN, HH), lambda c, g: (NBN + g, 0)),
                  _vspec((2, HH, HH)), _vspec((2, HH, HH)),
                  pl.BlockSpec((BN, 1), lambda c, g: (g, 0))],
        out_specs=[pl.BlockSpec((1, BL, HH), lambda c, g: (c, 0, 0)),
                   pl.BlockSpec((1, BL), lambda c, g: (0, 0))],
        out_shape=[jax.ShapeDtypeStruct((2, BL, HH), f32),
                   jax.ShapeDtypeStruct((1, BL), f32)],
        scratch_shapes=[pltpu.VMEM((BL, HH), f32), pltpu.VMEM((1, BL), f32)],
        cost_estimate=cost_big,
    )(hu2_st, agg2b_st, agg2b_st, wu2a_lo, wu2a_hi, batch2)

    shot = pl.pallas_call(
        _shot_k,
        in_specs=[_vspec((2, BL, HH)), _vspec((1, BL)),
                  _vspec((HH, H)), _vspec((HH, H)), _vspec((1, H)),
                  _vspec((H, 1)), _vspec((1, 1))],
        out_specs=_vspec((B, 1)),
        out_shape=jax.ShapeDtypeStruct((B, 1), f32),
    )(pooled, counts, ws1_lo, ws1_hi, bs1, ws2, bs2)

    return probs[:, 0], ind[:, 0], shot


# speedup vs baseline: 2.9746x; 1.0499x over previous
import functools

import jax
import jax.numpy as jnp
from jax import lax
from jax.experimental import pallas as pl
from jax.experimental.pallas import tpu as pltpu
from jax.experimental.pallas import tpu_sc as plsc

N = 50000
E = 800000
B = 1000
H = 64
HH = 32
BN = 1000
NBN = N // BN
BE = 2000
NBE = E // BE
BL = 1024
NEG = -1e30

NSUB = 16
EPT = E // NSUB
C = 200
NCHUNK = EPT // C
NPAIR = NCHUNK // 2
GS = 40
NG = C // GS
CPAD = 208
RZ = N // NSUB


def _msg_body(p_st, q_st, src_h, dst_h, agg_h,
              idx0, idx1, rows0, rows1, q0, q1, d0, d1,
              agg_sp, sin0, sin1, sg0, sg1, ssc0, ssc1):
    c = lax.axis_index("c")
    s = lax.axis_index("s")
    slots = ((idx0, rows0, q0, d0, sin0, sg0),
             (idx1, rows1, q1, d1, sin1, sg1))

    z16 = jnp.zeros((16,), jnp.float32)

    def zb(i, carry):
        rows0[i, pl.ds(0, 16)] = z16
        rows0[i, pl.ds(16, 16)] = z16
        return carry

    lax.fori_loop(0, C, zb, 0)
    for k in range(RZ // C):
        pltpu.sync_copy(rows0, agg_sp.at[pl.ds(s * RZ + k * C, C)])
    pltpu.sync_copy(rows0.at[pl.ds(0, RZ % C)],
                    agg_sp.at[pl.ds(s * RZ + (RZ // C) * C, RZ % C)])
    plsc.subcore_barrier()

    base0 = s * EPT
    coff = c * N

    def in_copies(ch, sl):
        idxb, rowsb, qb, dstb, sin, sg = sl
        base = base0 + ch * C
        return (
            pltpu.make_async_copy(src_h.at[pl.ds(base, C)],
                                  idxb.at[pl.ds(0, C)], sin),
            pltpu.make_async_copy(q_st.at[pl.ds(c * E + base, C)], qb, sin),
            pltpu.make_async_copy(dst_h.at[pl.ds(base // GS, NG)], dstb, sin),
        )

    def issue_in(ch, sl):
        for cp in in_copies(ch, sl):
            cp.start()

    def wait_in(ch, sl):
        for cp in in_copies(ch, sl):
            cp.wait()

    def gather_copies(sl):
        idxb, rowsb, qb, dstb, sin, sg = sl
        return tuple(
            pltpu.make_async_copy(p_st.at[idxb.at[pl.ds(j * GS, GS)]],
                                  rowsb.at[pl.ds(j * GS, GS)], sg)
            for j in range(NG))

    def idx_add_and_gather(sl):
        idxb = sl[0]
        for r in range(CPAD // 16):
            idxb[pl.ds(r * 16, 16)] = idxb[pl.ds(r * 16, 16)] + coff
        for cp in gather_copies(sl):
            cp.start()

    def compute_scatter(sl, ssc):
        idxb, rowsb, qb, dstb, sin, sg = sl
        for cp in gather_copies(sl):
            cp.wait()
        U = 8

        def mb(i, carry):
            for u in range(U):
                e = i * U + u
                a = rowsb[e, pl.ds(0, 16)] + qb[e, pl.ds(0, 16)]
                rowsb[e, pl.ds(0, 16)] = jnp.maximum(a, 0.0)
                b2 = rowsb[e, pl.ds(16, 16)] + qb[e, pl.ds(16, 16)]
                rowsb[e, pl.ds(16, 16)] = jnp.maximum(b2, 0.0)
            return carry

        lax.fori_loop(0, C // U, mb, 0)
        for j in range(NG):
            pltpu.async_copy(rowsb.at[pl.ds(j * GS, GS)],
                             agg_sp.at[dstb.at[j]], ssc, add=True)

    def wait_scatter(sl, ssc):
        idxb, rowsb, qb, dstb, sin, sg = sl
        for j in range(NG):
            pltpu.make_async_copy(rowsb.at[pl.ds(j * GS, GS)],
                                  agg_sp.at[dstb.at[j]], ssc).wait()

    issue_in(0, slots[0])

    def pair(g, carry):
        c0 = 2 * g
        issue_in(c0 + 1, slots[1])
        wait_in(c0, slots[0])

        @pl.when(g > 0)
        def _():
            wait_scatter(slots[0], ssc0)

        idx_add_and_gather(slots[0])
        wait_in(c0 + 1, slots[1])

        @pl.when(g > 0)
        def _():
            wait_scatter(slots[1], ssc1)

        idx_add_and_gather(slots[1])
        compute_scatter(slots[0], ssc0)

        @pl.when(g < NPAIR - 1)
        def _():
            issue_in(c0 + 2, slots[0])

        compute_scatter(slots[1], ssc1)
        return carry

    lax.fori_loop(0, NPAIR, pair, 0)
    wait_scatter(slots[0], ssc0)
    wait_scatter(slots[1], ssc1)
    plsc.subcore_barrier()
    pltpu.sync_copy(agg_sp.at[pl.ds(s * RZ, RZ)],
                    agg_h.at[pl.ds(coff + s * RZ, RZ)])


@functools.cache
def _get_msg_call():
  return pl.kernel(
    _msg_body,
    out_type=jax.ShapeDtypeStruct((2 * N, HH), jnp.float32),
    mesh=plsc.VectorSubcoreMesh(core_axis_name="c", subcore_axis_name="s"),
    compiler_params=pltpu.CompilerParams(use_tc_tiling_on_sc=False),
    scratch_types=[
        pltpu.VMEM((CPAD,), jnp.int32),
        pltpu.VMEM((CPAD,), jnp.int32),
        pltpu.VMEM((C, HH), jnp.float32),
        pltpu.VMEM((C, HH), jnp.float32),
        pltpu.VMEM((C, HH), jnp.float32),
        pltpu.VMEM((C, HH), jnp.float32),
        pltpu.VMEM((NG, GS), jnp.int32),
        pltpu.VMEM((NG, GS), jnp.int32),
        pltpu.VMEM_SHARED((N, HH), jnp.float32),
        pltpu.SemaphoreType.DMA,
        pltpu.SemaphoreType.DMA,
        pltpu.SemaphoreType.DMA,
        pltpu.SemaphoreType.DMA,
        pltpu.SemaphoreType.DMA,
        pltpu.SemaphoreType.DMA,
    ],
  )


def _sel(c, lo, hi):
    return jnp.where(c == 0, lo, hi)


def _pre_k(x, wmlo, wmhi, bmlo, bmhi, wulo, wuhi, bulo, buhi, p_st, xu_st):
    c = pl.program_id(0)
    xb = x[...]
    p_st[...] = jnp.dot(xb, _sel(c, wmlo[...], wmhi[...]),
                        preferred_element_type=jnp.float32) + _sel(
                            c, bmlo[...], bmhi[...])
    xu_st[...] = jnp.dot(xb, _sel(c, wulo[...], wuhi[...]),
                         preferred_element_type=jnp.float32) + _sel(
                             c, bulo[...], buhi[...])


def _qtab_k(attr, w1lo, w1hi, w2lo, w2hi, q1_st, q2_st):
    c = pl.program_id(0)
    ab = attr[...]
    q1_st[...] = jnp.dot(ab, _sel(c, w1lo[...], w1hi[...]),
                         preferred_element_type=jnp.float32)
    q2_st[...] = jnp.dot(ab, _sel(c, w2lo[...], w2hi[...]),
                         preferred_element_type=jnp.float32)


def _upd_k(base_st, agg_lo, agg_hi, wa_lo, wa_hi, wb_lo, wb_hi, h_st):
    c = pl.program_id(0)
    acc = base_st[...]
    acc += jnp.dot(agg_lo[...], _sel(c, wa_lo[...], wa_hi[...]),
                   preferred_element_type=jnp.float32)
    acc += jnp.dot(agg_hi[...], _sel(c, wb_lo[...], wb_hi[...]),
                   preferred_element_type=jnp.float32)
    h_st[...] = jnp.maximum(acc, 0.0)


def _tab2_k(h_lo, h_hi, wa_ll, wa_lh, wa_hl, wa_hh, ba_lo, ba_hi,
            wb_ll, wb_lh, wb_hl, wb_hh, bb_lo, bb_hi, a_st, b_st):
    c = pl.program_id(0)
    hl = h_lo[...]
    hh = h_hi[...]
    a_st[...] = (jnp.dot(hl, _sel(c, wa_ll[...], wa_lh[...]),
                         preferred_element_type=jnp.float32)
                 + jnp.dot(hh, _sel(c, wa_hl[...], wa_hh[...]),
                           preferred_element_type=jnp.float32)
                 + _sel(c, ba_lo[...], ba_hi[...]))
    b_st[...] = (jnp.dot(hl, _sel(c, wb_ll[...], wb_lh[...]),
                         preferred_element_type=jnp.float32)
                 + jnp.dot(hh, _sel(c, wb_hl[...], wb_hh[...]),
                           preferred_element_type=jnp.float32)
                 + _sel(c, bb_lo[...], bb_hi[...]))


def _logits_k(hu_st, agg_lo, agg_hi, wlo, whi, wr_lo, wr_hi, br, logits):
    h2lo = jnp.maximum(
        hu_st[0] + jnp.dot(agg_lo[...], wlo[0],
                           preferred_element_type=jnp.float32)
        + jnp.dot(agg_hi[...], whi[0], preferred_element_type=jnp.float32),
        0.0)
    h2hi = jnp.maximum(
        hu_st[1] + jnp.dot(agg_lo[...], wlo[1],
                           preferred_element_type=jnp.float32)
        + jnp.dot(agg_hi[...], whi[1], preferred_element_type=jnp.float32),
        0.0)
    lg = (jnp.dot(h2lo, wr_lo[...], preferred_element_type=jnp.float32)
          + jnp.dot(h2hi, wr_hi[...], preferred_element_type=jnp.float32)
          + br[...])
    logits[...] = lg


def _segmax_k(logits, maskf, batch, m_out, macc):
    g = pl.program_id(0)

    @pl.when(g == 0)
    def _():
        macc[...] = jnp.full((1, BL), -jnp.inf, jnp.float32)

    oh = batch[...] == lax.broadcasted_iota(jnp.int32, (BN, BL), 1)
    ml = jnp.where(maskf[...] > 0, logits[...], NEG)
    mx = jnp.max(jnp.where(oh, ml, -jnp.inf), axis=0, keepdims=True)
    macc[...] = jnp.maximum(macc[...], mx)

    @pl.when(g == NBN - 1)
    def _():
        mm = macc[...]
        m_out[...] = jnp.where(jnp.isfinite(mm), mm, 0.0)


def _exp_k(logits, maskf, batch, m, e_out, denom, dacc):
    g = pl.program_id(0)

    @pl.when(g == 0)
    def _():
        dacc[...] = jnp.zeros((1, BL), jnp.float32)

    oh = batch[...] == lax.broadcasted_iota(jnp.int32, (BN, BL), 1)
    mg = jnp.sum(jnp.where(oh, m[...], 0.0), axis=1, keepdims=True)
    z = jnp.where(maskf[...] > 0, logits[...] - mg, NEG)
    e = jnp.exp(z)
    e_out[...] = e
    dacc[...] += jnp.sum(jnp.where(oh, e, 0.0), axis=0, keepdims=True)

    @pl.when(g == NBN - 1)
    def _():
        denom[...] = dacc[...]


def _probs_k(e_in, batch, denom, probs, pm_out, ss_out, pmacc, ssacc):
    g = pl.program_id(0)

    @pl.when(g == 0)
    def _():
        pmacc[...] = jnp.full((1, BL), -jnp.inf, jnp.float32)
        ssacc[...] = jnp.zeros((1, BL), jnp.float32)

    oh = batch[...] == lax.broadcasted_iota(jnp.int32, (BN, BL), 1)
    dg = jnp.sum(jnp.where(oh, denom[...], 0.0), axis=1, keepdims=True)
    p = e_in[...] / (dg + 1e-12)
    probs[...] = p
    pmacc[...] = jnp.maximum(
        pmacc[...], jnp.max(jnp.where(oh, p, -jnp.inf), axis=0,
                            keepdims=True))
    ssacc[...] += jnp.sum(jnp.where(oh, p, 0.0), axis=0, keepdims=True)

    @pl.when(g == NBN - 1)
    def _():
        pm_out[...] = pmacc[...]
        ss_out[...] = ssacc[...]


def _first_k(probs, batch, pm, ss, tgt_out, facc):
    g = pl.program_id(0)
    imax = jnp.int32(2147483647)

    @pl.when(g == 0)
    def _():
        facc[...] = jnp.full((1, BL), imax, jnp.int32)

    oh = batch[...] == lax.broadcasted_iota(jnp.int32, (BN, BL), 1)
    pg = jnp.sum(jnp.where(oh, pm[...], 0.0), axis=1, keepdims=True)
    idxv = (g * BN
            + lax.broadcasted_iota(jnp.int32, (BN, 1), 0))
    cand = jnp.where(probs[...] == pg, idxv, jnp.int32(N))
    cmin = jnp.min(jnp.where(oh, cand, imax), axis=0, keepdims=True)
    facc[...] = jnp.minimum(facc[...], cmin)

    @pl.when(g == NBN - 1)
    def _():
        tgt_out[...] = jnp.where((ss[...] > 0) & (facc[...] < N),
                                 facc[...], jnp.int32(N))


def _fix_k(batch, tgt, p1_st, xu_st, wm_lo, wm_hi, wu_lo, wu_hi,
           ind_out, p1p_st, xup_st):
    c = pl.program_id(0)
    g = pl.program_id(1)
    oh = batch[...] == lax.broadcasted_iota(jnp.int32, (BN, BL), 1)
    tg = jnp.sum(jnp.where(oh, tgt[...], 0), axis=1, keepdims=True)
    idxv = g * BN + lax.broadcasted_iota(jnp.int32, (BN, 1), 0)
    ind = jnp.where(idxv == tg, 1.0, 0.0).astype(jnp.float32)
    ind_out[...] = ind
    p1p_st[...] = p1_st[...] + ind * _sel(c, wm_lo[...], wm_hi[...])
    xup_st[...] = xu_st[...] + ind * _sel(c, wu_lo[...], wu_hi[...])


def _pool_k(hu_st, agg_lo, agg_hi, wlo, whi, batch, pooled, counts,
            pacc, cacc):
    c = pl.program_id(0)
    g = pl.program_id(1)

    @pl.when(g == 0)
    def _():
        pacc[...] = jnp.zeros((BL, HH), jnp.float32)
        cacc[...] = jnp.zeros((1, BL), jnp.float32)

    h2 = jnp.maximum(
        hu_st[...] + jnp.dot(agg_lo[...], _sel(c, wlo[0], wlo[1]),
                             preferred_element_type=jnp.float32)
        + jnp.dot(agg_hi[...], _sel(c, whi[0], whi[1]),
                  preferred_element_type=jnp.float32), 0.0)
    ohf = (batch[...] == lax.broadcasted_iota(jnp.int32, (BN, BL), 1)
           ).astype(jnp.float32)
    pacc[...] += lax.dot_general(ohf, h2, (((0,), (0,)), ((), ())),
                                 preferred_element_type=jnp.float32,
                                 precision=lax.Precision.HIGHEST)
    cacc[...] += jnp.sum(ohf, axis=0, keepdims=True)

    @pl.when(g == NBN - 1)
    def _():
        pooled[0] = pacc[...]
        counts[...] = cacc[...]


def _shot_k(pooled, counts, w1_lo, w1_hi, b1, w2, b2, shot):
    cnt = jnp.maximum(counts[...], 1.0)
    inv = (1.0 / cnt).reshape(BL, 1)
    emb_lo = pooled[0] * inv
    emb_hi = pooled[1] * inv
    s = jnp.maximum(
        jnp.dot(emb_lo, w1_lo[...], preferred_element_type=jnp.float32)
        + jnp.dot(emb_hi, w1_hi[...], preferred_element_type=jnp.float32)
        + b1[...], 0.0)
    lg = jnp.dot(s, w2[...], preferred_element_type=jnp.float32) + b2[...]
    shot[...] = lg[:B, :]


def _vspec(shape):
    return pl.BlockSpec(shape, lambda *args: tuple(0 for _ in shape))


def kernel(x, edge_index, edge_attr, receiver_mask, batch,
           W_msg1, b_msg1, W_upd1, b_upd1, W_msg2, b_msg2, W_upd2, b_upd2,
           W_recv, b_recv, W_shot1, b_shot1, W_shot2, b_shot2):
    f32 = jnp.float32
    src = edge_index[0]
    dst = edge_index[1]
    dst2d = dst.reshape(E // GS, GS)
    maskf = receiver_mask.astype(f32).reshape(N, 1)
    batch2 = batch.reshape(N, 1)

    wm1x_lo, wm1x_hi = W_msg1[:13, :HH], W_msg1[:13, HH:]
    _b16 = lambda w: w.astype(jnp.bfloat16).astype(jnp.float32)
    wm1i_lo, wm1i_hi = _b16(W_msg1[13:14, :HH]), _b16(W_msg1[13:14, HH:])
    wm1a_lo, wm1a_hi = W_msg1[14:18, :HH], W_msg1[14:18, HH:]
    bm1_lo, bm1_hi = b_msg1.reshape(1, H)[:, :HH], b_msg1.reshape(1, H)[:, HH:]
    wu1x_lo, wu1x_hi = W_upd1[:13, :HH], W_upd1[:13, HH:]
    wu1i_lo, wu1i_hi = _b16(W_upd1[13:14, :HH]), _b16(W_upd1[13:14, HH:])
    wu1a = W_upd1[14:78]
    wu1a_ll, wu1a_lh = wu1a[:HH, :HH], wu1a[:HH, HH:]
    wu1a_hl, wu1a_hh = wu1a[HH:, :HH], wu1a[HH:, HH:]
    bu1_lo, bu1_hi = b_upd1.reshape(1, H)[:, :HH], b_upd1.reshape(1, H)[:, HH:]
    wm2x = W_msg2[:64]
    wm2a_lo, wm2a_hi = W_msg2[64:68, :HH], W_msg2[64:68, HH:]
    bm2 = b_msg2.reshape(1, H)
    wu2h = W_upd2[:64]
    wu2a = W_upd2[64:128]
    bu2 = b_upd2.reshape(1, H)
    wu2a_lo = jnp.stack([wu2a[:HH, :HH], wu2a[:HH, HH:]])
    wu2a_hi = jnp.stack([wu2a[HH:, :HH], wu2a[HH:, HH:]])
    wr_lo = W_recv[:HH]
    wr_hi = W_recv[HH:]
    br = b_recv.reshape(1, 1)
    ws1_lo, ws1_hi = W_shot1[:HH], W_shot1[HH:]
    bs1 = b_shot1.reshape(1, H)
    ws2 = W_shot2
    bs2 = b_shot2.reshape(1, 1)

    cost_big = pl.CostEstimate(flops=2 * N * 78 * H, bytes_accessed=N * 600,
                               transcendentals=0)

    wspec13 = _vspec((13, HH))
    bspec = _vspec((1, HH))
    p1_st, xu_st = pl.pallas_call(
        _pre_k,
        grid=(2, NBN),
        in_specs=[pl.BlockSpec((BN, 13), lambda c, g: (g, 0))] +
                 [wspec13, wspec13, bspec, bspec, wspec13, wspec13, bspec,
                  bspec],
        out_specs=[pl.BlockSpec((BN, HH), lambda c, g: (c * NBN + g, 0))] * 2,
        out_shape=[jax.ShapeDtypeStruct((2 * N, HH), f32)] * 2,
    )(x, wm1x_lo, wm1x_hi, bm1_lo, bm1_hi, wu1x_lo, wu1x_hi, bu1_lo, bu1_hi)

    wspec4 = _vspec((4, HH))
    q1_st, q2_st = pl.pallas_call(
        _qtab_k,
        grid=(2, NBE),
        in_specs=[pl.BlockSpec((BE, 4), lambda c, g: (g, 0)),
                  wspec4, wspec4, wspec4, wspec4],
        out_specs=[pl.BlockSpec((BE, HH), lambda c, g: (c * NBE + g, 0))] * 2,
        out_shape=[jax.ShapeDtypeStruct((2 * E, HH), f32)] * 2,
    )(edge_attr, wm1a_lo, wm1a_hi, wm2a_lo, wm2a_hi)

    wspecH = _vspec((HH, HH))

    def upd(base_st, agg_st, wll, wlh, whl, whh):
        return pl.pallas_call(
            _upd_k,
            grid=(2, NBN),
            in_specs=[pl.BlockSpec((BN, HH), lambda c, g: (c * NBN + g, 0)),
                      pl.BlockSpec((BN, HH), lambda c, g: (g, 0)),
                      pl.BlockSpec((BN, HH), lambda c, g: (NBN + g, 0)),
                      wspecH, wspecH, wspecH, wspecH],
            out_specs=pl.BlockSpec((BN, HH), lambda c, g: (c * NBN + g, 0)),
            out_shape=jax.ShapeDtypeStruct((2 * N, HH), f32),
            cost_estimate=cost_big,
        )(base_st, agg_st, agg_st, wll, wlh, whl, whh)

    def tab2(h_st, WA, bA, WB, bB):
        return pl.pallas_call(
            _tab2_k,
            grid=(2, NBN),
            in_specs=[pl.BlockSpec((BN, HH), lambda c, g: (g, 0)),
                      pl.BlockSpec((BN, HH), lambda c, g: (NBN + g, 0)),
                      wspecH, wspecH, wspecH, wspecH, bspec, bspec,
                      wspecH, wspecH, wspecH, wspecH, bspec, bspec],
            out_specs=[pl.BlockSpec((BN, HH),
                                    lambda c, g: (c * NBN + g, 0))] * 2,
            out_shape=[jax.ShapeDtypeStruct((2 * N, HH), f32)] * 2,
            cost_estimate=cost_big,
        )(h_st, h_st,
          WA[:HH, :HH], WA[:HH, HH:], WA[HH:, :HH], WA[HH:, HH:],
          bA[:, :HH], bA[:, HH:],
          WB[:HH, :HH], WB[:HH, HH:], WB[HH:, :HH], WB[HH:, HH:],
          bB[:, :HH], bB[:, HH:])

    def backbone_tail(p1t, xut):
        msg = _get_msg_call()
        agg1_st = msg(p1t, q1_st, src, dst2d)
        h1_st = upd(xut, agg1_st, wu1a_ll, wu1a_lh, wu1a_hl, wu1a_hh)
        p2_st, hu_st = tab2(h1_st, wm2x, bm2, wu2h, bu2)
        agg2_st = msg(p2_st, q2_st, src, dst2d)
        return hu_st, agg2_st

    hu_st, agg2_st = backbone_tail(p1_st, xu_st)
    logits = pl.pallas_call(
        _logits_k,
        grid=(NBN,),
        in_specs=[pl.BlockSpec((2, BN, HH), lambda g: (0, g, 0)),
                  pl.BlockSpec((BN, HH), lambda g: (g, 0)),
                  pl.BlockSpec((BN, HH), lambda g: (NBN + g, 0)),
                  _vspec((2, HH, HH)), _vspec((2, HH, HH)),
                  _vspec((HH, 1)), _vspec((HH, 1)), _vspec((1, 1))],
        out_specs=pl.BlockSpec((BN, 1), lambda g: (g, 0)),
        out_shape=jax.ShapeDtypeStruct((N, 1), f32),
        cost_estimate=cost_big,
    )(hu_st.reshape(2, N, HH), agg2_st, agg2_st, wu2a_lo, wu2a_hi,
      wr_lo, wr_hi, br)

    nspec = pl.BlockSpec((BN, 1), lambda g: (g, 0))
    bl_spec = _vspec((1, BL))
    segargs = dict(grid=(NBN,))
    m_seg = pl.pallas_call(
        _segmax_k, in_specs=[nspec, nspec, nspec], out_specs=bl_spec,
        out_shape=jax.ShapeDtypeStruct((1, BL), f32),
        scratch_shapes=[pltpu.VMEM((1, BL), f32)], **segargs,
    )(logits, maskf, batch2)
    e_arr, denom = pl.pallas_call(
        _exp_k, in_specs=[nspec, nspec, nspec, bl_spec],
        out_specs=[nspec, bl_spec],
        out_shape=[jax.ShapeDtypeStruct((N, 1), f32),
                   jax.ShapeDtypeStruct((1, BL), f32)],
        scratch_shapes=[pltpu.VMEM((1, BL), f32)], **segargs,
    )(logits, maskf, batch2, m_seg)
    probs, pm, ss = pl.pallas_call(
        _probs_k, in_specs=[nspec, nspec, bl_spec],
        out_specs=[nspec, bl_spec, bl_spec],
        out_shape=[jax.ShapeDtypeStruct((N, 1), f32),
                   jax.ShapeDtypeStruct((1, BL), f32),
                   jax.ShapeDtypeStruct((1, BL), f32)],
        scratch_shapes=[pltpu.VMEM((1, BL), f32)] * 2, **segargs,
    )(e_arr, batch2, denom)
    tgt = pl.pallas_call(
        _first_k, in_specs=[nspec, nspec, bl_spec, bl_spec],
        out_specs=pl.BlockSpec((1, BL), lambda g: (0, 0)),
        out_shape=jax.ShapeDtypeStruct((1, BL), jnp.int32),
        scratch_shapes=[pltpu.VMEM((1, BL), jnp.int32)], **segargs,
    )(probs, batch2, pm, ss)

    wspec1 = _vspec((1, HH))
    ind, p1p_st, xup_st = pl.pallas_call(
        _fix_k,
        grid=(2, NBN),
        in_specs=[pl.BlockSpec((BN, 1), lambda c, g: (g, 0)),
                  pl.BlockSpec((1, BL), lambda c, g: (0, 0)),
                  pl.BlockSpec((BN, HH), lambda c, g: (c * NBN + g, 0)),
                  pl.BlockSpec((BN, HH), lambda c, g: (c * NBN + g, 0)),
                  wspec1, wspec1, wspec1, wspec1],
        out_specs=[pl.BlockSpec((BN, 1), lambda c, g: (g, 0)),
                   pl.BlockSpec((BN, HH), lambda c, g: (c * NBN + g, 0)),
                   pl.BlockSpec((BN, HH), lambda c, g: (c * NBN + g, 0))],
        out_shape=[jax.ShapeDtypeStruct((N, 1), f32),
                   jax.ShapeDtypeStruct((2 * N, HH), f32),
                   jax.ShapeDtypeStruct((2 * N, HH), f32)],
    )(batch2, tgt, p1_st, xu_st, wm1i_lo, wm1i_hi, wu1i_lo, wu1i_hi)

    hu2_st, agg2b_st = backbone_tail(p1p_st, xup_st)
    pooled, counts = pl.pallas_call(
        _pool_k,
        grid=(2, NBN),
        in_specs=[pl.BlockSpec((BN, HH), lambda c, g: (c * NBN + g, 0)),
                  pl.BlockSpec((BN, HH), lambda c, g: (g, 0)),
                  pl.BlockSpec((BN, HH), lambda c, g: (NBN + g, 0)),
                  _vspec((2, HH, HH)), _vspec((2, HH, HH)),
                  pl.BlockSpec((BN, 1), lambda c, g: (g, 0))],
        out_specs=[pl.BlockSpec((1, BL, HH), lambda c, g: (c, 0, 0)),
                   pl.BlockSpec((1, BL), lambda c, g: (0, 0))],
        out_shape=[jax.ShapeDtypeStruct((2, BL, HH), f32),
                   jax.ShapeDtypeStruct((1, BL), f32)],
        scratch_shapes=[pltpu.VMEM((BL, HH), f32), pltpu.VMEM((1, BL), f32)],
        cost_estimate=cost_big,
    )(hu2_st, agg2b_st, agg2b_st, wu2a_lo, wu2a_hi, batch2)

    shot = pl.pallas_call(
        _shot_k,
        in_specs=[_vspec((2, BL, HH)), _vspec((1, BL)),
                  _vspec((HH, H)), _vspec((HH, H)), _vspec((1, H)),
                  _vspec((H, 1)), _vspec((1, 1))],
        out_specs=_vspec((B, 1)),
        out_shape=jax.ShapeDtypeStruct((B, 1), f32),
    )(pooled, counts, ws1_lo, ws1_hi, bs1, ws2, bs2)

    return probs[:, 0], ind[:, 0], shot


# speedup vs baseline: 3.2429x; 1.0902x over previous
import functools

import jax
import jax.numpy as jnp
from jax import lax
from jax.experimental import pallas as pl
from jax.experimental.pallas import tpu as pltpu
from jax.experimental.pallas import tpu_sc as plsc

N = 50000
E = 800000
B = 1000
H = 64
HH = 32
BN = 1000
NBN = N // BN
BE = 2000
NBE = E // BE
BL = 1024
NEG = -1e30

NSUB = 16
EPT = E // NSUB
C = 200
NCHUNK = EPT // C
NPAIR = NCHUNK // 2
GS = 40
NG = C // GS
CPAD = 208
RZ = N // NSUB


def _msg_body(p_st, q_st, src_h, dst_h, agg_h,
              idx0, idx1, rows0, rows1, q0, q1, d0, d1,
              agg_sp, sin0, sin1, sg0, sg1, ssc0, ssc1):
    c = lax.axis_index("c")
    s = lax.axis_index("s")
    slots = ((idx0, rows0, q0, d0, sin0, sg0),
             (idx1, rows1, q1, d1, sin1, sg1))

    z16 = jnp.zeros((16,), jnp.float32)

    def zb(i, carry):
        rows0[i, pl.ds(0, 16)] = z16
        rows0[i, pl.ds(16, 16)] = z16
        return carry

    lax.fori_loop(0, C, zb, 0)
    for k in range(RZ // C):
        pltpu.sync_copy(rows0, agg_sp.at[pl.ds(s * RZ + k * C, C)])
    pltpu.sync_copy(rows0.at[pl.ds(0, RZ % C)],
                    agg_sp.at[pl.ds(s * RZ + (RZ // C) * C, RZ % C)])
    plsc.subcore_barrier()

    base0 = s * EPT
    coff = c * N

    def in_copies(ch, sl):
        idxb, rowsb, qb, dstb, sin, sg = sl
        base = base0 + ch * C
        return (
            pltpu.make_async_copy(src_h.at[pl.ds(base, C)],
                                  idxb.at[pl.ds(0, C)], sin),
            pltpu.make_async_copy(
                q_st.at[pl.ds(c * (E // 4) + base // 4, C // 4)], qb, sin),
            pltpu.make_async_copy(dst_h.at[pl.ds(base // GS, NG)], dstb, sin),
        )

    def issue_in(ch, sl):
        for cp in in_copies(ch, sl):
            cp.start()

    def wait_in(ch, sl):
        for cp in in_copies(ch, sl):
            cp.wait()

    def gather_copies(sl):
        idxb, rowsb, qb, dstb, sin, sg = sl
        return tuple(
            pltpu.make_async_copy(p_st.at[idxb.at[pl.ds(j * GS, GS)]],
                                  rowsb.at[pl.ds(j * GS, GS)], sg)
            for j in range(NG))

    def idx_add_and_gather(sl):
        idxb = sl[0]
        for r in range(CPAD // 16):
            idxb[pl.ds(r * 16, 16)] = idxb[pl.ds(r * 16, 16)] + coff
        for cp in gather_copies(sl):
            cp.start()

    def compute_scatter(sl, ssc):
        idxb, rowsb, qb, dstb, sin, sg = sl
        for cp in gather_copies(sl):
            cp.wait()
        U = 8

        def mb(i, carry):
            for u in range(U):
                e = i * U + u
                qr = 2 * i + (u // 4)
                qc = (u % 4) * 32
                a = rowsb[e, pl.ds(0, 16)] + qb[qr, pl.ds(qc, 16)]
                rowsb[e, pl.ds(0, 16)] = jnp.maximum(a, 0.0)
                b2 = rowsb[e, pl.ds(16, 16)] + qb[qr, pl.ds(qc + 16, 16)]
                rowsb[e, pl.ds(16, 16)] = jnp.maximum(b2, 0.0)
            return carry

        lax.fori_loop(0, C // U, mb, 0)
        for j in range(NG):
            pltpu.async_copy(rowsb.at[pl.ds(j * GS, GS)],
                             agg_sp.at[dstb.at[j]], ssc, add=True)

    def wait_scatter(sl, ssc):
        idxb, rowsb, qb, dstb, sin, sg = sl
        for j in range(NG):
            pltpu.make_async_copy(rowsb.at[pl.ds(j * GS, GS)],
                                  agg_sp.at[dstb.at[j]], ssc).wait()

    issue_in(0, slots[0])

    def pair(g, carry):
        c0 = 2 * g
        issue_in(c0 + 1, slots[1])
        wait_in(c0, slots[0])

        @pl.when(g > 0)
        def _():
            wait_scatter(slots[0], ssc0)

        idx_add_and_gather(slots[0])
        wait_in(c0 + 1, slots[1])

        @pl.when(g > 0)
        def _():
            wait_scatter(slots[1], ssc1)

        idx_add_and_gather(slots[1])
        compute_scatter(slots[0], ssc0)

        @pl.when(g < NPAIR - 1)
        def _():
            issue_in(c0 + 2, slots[0])

        compute_scatter(slots[1], ssc1)
        return carry

    lax.fori_loop(0, NPAIR, pair, 0)
    wait_scatter(slots[0], ssc0)
    wait_scatter(slots[1], ssc1)
    plsc.subcore_barrier()
    pltpu.sync_copy(agg_sp.at[pl.ds(s * RZ, RZ)],
                    agg_h.at[pl.ds(coff + s * RZ, RZ)])


@functools.cache
def _get_msg_call():
  return pl.kernel(
    _msg_body,
    out_type=jax.ShapeDtypeStruct((2 * N, HH), jnp.float32),
    mesh=plsc.VectorSubcoreMesh(core_axis_name="c", subcore_axis_name="s"),
    compiler_params=pltpu.CompilerParams(use_tc_tiling_on_sc=False),
    scratch_types=[
        pltpu.VMEM((CPAD,), jnp.int32),
        pltpu.VMEM((CPAD,), jnp.int32),
        pltpu.VMEM((C, HH), jnp.float32),
        pltpu.VMEM((C, HH), jnp.float32),
        pltpu.VMEM((C // 4, 4 * HH), jnp.float32),
        pltpu.VMEM((C // 4, 4 * HH), jnp.float32),
        pltpu.VMEM((NG, GS), jnp.int32),
        pltpu.VMEM((NG, GS), jnp.int32),
        pltpu.VMEM_SHARED((N, HH), jnp.float32),
        pltpu.SemaphoreType.DMA,
        pltpu.SemaphoreType.DMA,
        pltpu.SemaphoreType.DMA,
        pltpu.SemaphoreType.DMA,
        pltpu.SemaphoreType.DMA,
        pltpu.SemaphoreType.DMA,
    ],
  )


def _sel(c, lo, hi):
    return jnp.where(c == 0, lo, hi)


def _pre_k(x, wmlo, wmhi, bmlo, bmhi, wulo, wuhi, bulo, buhi, p_st, xu_st):
    c = pl.program_id(0)
    xb = x[...]
    p_st[...] = jnp.dot(xb, _sel(c, wmlo[...], wmhi[...]),
                        preferred_element_type=jnp.float32) + _sel(
                            c, bmlo[...], bmhi[...])
    xu_st[...] = jnp.dot(xb, _sel(c, wulo[...], wuhi[...]),
                         preferred_element_type=jnp.float32) + _sel(
                             c, bulo[...], buhi[...])


def _qtab_k(attr_pk, w1lo, w1hi, w2lo, w2hi, q1_pk, q2_pk):
    c = pl.program_id(0)
    ab = attr_pk[...]
    q1_pk[...] = jnp.dot(ab, _sel(c, w1lo[...], w1hi[...]),
                         preferred_element_type=jnp.float32)
    q2_pk[...] = jnp.dot(ab, _sel(c, w2lo[...], w2hi[...]),
                         preferred_element_type=jnp.float32)


def _upd_k(base_st, agg_lo, agg_hi, wa_lo, wa_hi, wb_lo, wb_hi, h_st):
    c = pl.program_id(0)
    acc = base_st[...]
    acc += jnp.dot(agg_lo[...], _sel(c, wa_lo[...], wa_hi[...]),
                   preferred_element_type=jnp.float32)
    acc += jnp.dot(agg_hi[...], _sel(c, wb_lo[...], wb_hi[...]),
                   preferred_element_type=jnp.float32)
    h_st[...] = jnp.maximum(acc, 0.0)


def _tab2_k(h_lo, h_hi, wa_ll, wa_lh, wa_hl, wa_hh, ba_lo, ba_hi,
            wb_ll, wb_lh, wb_hl, wb_hh, bb_lo, bb_hi, a_st, b_st):
    c = pl.program_id(0)
    hl = h_lo[...]
    hh = h_hi[...]
    a_st[...] = (jnp.dot(hl, _sel(c, wa_ll[...], wa_lh[...]),
                         preferred_element_type=jnp.float32)
                 + jnp.dot(hh, _sel(c, wa_hl[...], wa_hh[...]),
                           preferred_element_type=jnp.float32)
                 + _sel(c, ba_lo[...], ba_hi[...]))
    b_st[...] = (jnp.dot(hl, _sel(c, wb_ll[...], wb_lh[...]),
                         preferred_element_type=jnp.float32)
                 + jnp.dot(hh, _sel(c, wb_hl[...], wb_hh[...]),
                           preferred_element_type=jnp.float32)
                 + _sel(c, bb_lo[...], bb_hi[...]))


def _logits_k(hu_st, agg_lo, agg_hi, wlo, whi, wr_lo, wr_hi, br, logits):
    h2lo = jnp.maximum(
        hu_st[0] + jnp.dot(agg_lo[...], wlo[0],
                           preferred_element_type=jnp.float32)
        + jnp.dot(agg_hi[...], whi[0], preferred_element_type=jnp.float32),
        0.0)
    h2hi = jnp.maximum(
        hu_st[1] + jnp.dot(agg_lo[...], wlo[1],
                           preferred_element_type=jnp.float32)
        + jnp.dot(agg_hi[...], whi[1], preferred_element_type=jnp.float32),
        0.0)
    lg = (jnp.dot(h2lo, wr_lo[...], preferred_element_type=jnp.float32)
          + jnp.dot(h2hi, wr_hi[...], preferred_element_type=jnp.float32)
          + br[...])
    logits[...] = lg


def _segmax_k(logits, maskf, batch, m_out, macc):
    g = pl.program_id(0)

    @pl.when(g == 0)
    def _():
        macc[...] = jnp.full((1, BL), -jnp.inf, jnp.float32)

    oh = batch[...] == lax.broadcasted_iota(jnp.int32, (BN, BL), 1)
    ml = jnp.where(maskf[...] > 0, logits[...], NEG)
    mx = jnp.max(jnp.where(oh, ml, -jnp.inf), axis=0, keepdims=True)
    macc[...] = jnp.maximum(macc[...], mx)

    @pl.when(g == NBN - 1)
    def _():
        mm = macc[...]
        m_out[...] = jnp.where(jnp.isfinite(mm), mm, 0.0)


def _exp_k(logits, maskf, batch, m, e_out, denom, dacc):
    g = pl.program_id(0)

    @pl.when(g == 0)
    def _():
        dacc[...] = jnp.zeros((1, BL), jnp.float32)

    oh = batch[...] == lax.broadcasted_iota(jnp.int32, (BN, BL), 1)
    mg = jnp.sum(jnp.where(oh, m[...], 0.0), axis=1, keepdims=True)
    z = jnp.where(maskf[...] > 0, logits[...] - mg, NEG)
    e = jnp.exp(z)
    e_out[...] = e
    dacc[...] += jnp.sum(jnp.where(oh, e, 0.0), axis=0, keepdims=True)

    @pl.when(g == NBN - 1)
    def _():
        denom[...] = dacc[...]


def _probs_k(e_in, batch, denom, probs, pm_out, ss_out, pmacc, ssacc):
    g = pl.program_id(0)

    @pl.when(g == 0)
    def _():
        pmacc[...] = jnp.full((1, BL), -jnp.inf, jnp.float32)
        ssacc[...] = jnp.zeros((1, BL), jnp.float32)

    oh = batch[...] == lax.broadcasted_iota(jnp.int32, (BN, BL), 1)
    dg = jnp.sum(jnp.where(oh, denom[...], 0.0), axis=1, keepdims=True)
    p = e_in[...] / (dg + 1e-12)
    probs[...] = p
    pmacc[...] = jnp.maximum(
        pmacc[...], jnp.max(jnp.where(oh, p, -jnp.inf), axis=0,
                            keepdims=True))
    ssacc[...] += jnp.sum(jnp.where(oh, p, 0.0), axis=0, keepdims=True)

    @pl.when(g == NBN - 1)
    def _():
        pm_out[...] = pmacc[...]
        ss_out[...] = ssacc[...]


def _first_k(probs, batch, pm, ss, tgt_out, facc):
    g = pl.program_id(0)
    imax = jnp.int32(2147483647)

    @pl.when(g == 0)
    def _():
        facc[...] = jnp.full((1, BL), imax, jnp.int32)

    oh = batch[...] == lax.broadcasted_iota(jnp.int32, (BN, BL), 1)
    pg = jnp.sum(jnp.where(oh, pm[...], 0.0), axis=1, keepdims=True)
    idxv = (g * BN
            + lax.broadcasted_iota(jnp.int32, (BN, 1), 0))
    cand = jnp.where(probs[...] == pg, idxv, jnp.int32(N))
    cmin = jnp.min(jnp.where(oh, cand, imax), axis=0, keepdims=True)
    facc[...] = jnp.minimum(facc[...], cmin)

    @pl.when(g == NBN - 1)
    def _():
        tgt_out[...] = jnp.where((ss[...] > 0) & (facc[...] < N),
                                 facc[...], jnp.int32(N))


def _fix_k(batch, tgt, p1_st, xu_st, wm_lo, wm_hi, wu_lo, wu_hi,
           ind_out, p1p_st, xup_st):
    c = pl.program_id(0)
    g = pl.program_id(1)
    oh = batch[...] == lax.broadcasted_iota(jnp.int32, (BN, BL), 1)
    tg = jnp.sum(jnp.where(oh, tgt[...], 0), axis=1, keepdims=True)
    idxv = g * BN + lax.broadcasted_iota(jnp.int32, (BN, 1), 0)
    ind = jnp.where(idxv == tg, 1.0, 0.0).astype(jnp.float32)
    ind_out[...] = ind
    p1p_st[...] = p1_st[...] + ind * _sel(c, wm_lo[...], wm_hi[...])
    xup_st[...] = xu_st[...] + ind * _sel(c, wu_lo[...], wu_hi[...])


def _pool_k(hu_st, agg_lo, agg_hi, wlo, whi, batch, pooled, counts,
            pacc, cacc):
    c = pl.program_id(0)
    g = pl.program_id(1)

    @pl.when(g == 0)
    def _():
        pacc[...] = jnp.zeros((BL, HH), jnp.float32)
        cacc[...] = jnp.zeros((1, BL), jnp.float32)

    h2 = jnp.maximum(
        hu_st[...] + jnp.dot(agg_lo[...], _sel(c, wlo[0], wlo[1]),
                             preferred_element_type=jnp.float32)
        + jnp.dot(agg_hi[...], _sel(c, whi[0], whi[1]),
                  preferred_element_type=jnp.float32), 0.0)
    ohf = (batch[...] == lax.broadcasted_iota(jnp.int32, (BN, BL), 1)
           ).astype(jnp.float32)
    pacc[...] += lax.dot_general(ohf, h2, (((0,), (0,)), ((), ())),
                                 preferred_element_type=jnp.float32,
                                 precision=lax.Precision.HIGHEST)
    cacc[...] += jnp.sum(ohf, axis=0, keepdims=True)

    @pl.when(g == NBN - 1)
    def _():
        pooled[0] = pacc[...]
        counts[...] = cacc[...]


def _shot_k(pooled, counts, w1_lo, w1_hi, b1, w2, b2, shot):
    cnt = jnp.maximum(counts[...], 1.0)
    inv = (1.0 / cnt).reshape(BL, 1)
    emb_lo = pooled[0] * inv
    emb_hi = pooled[1] * inv
    s = jnp.maximum(
        jnp.dot(emb_lo, w1_lo[...], preferred_element_type=jnp.float32)
        + jnp.dot(emb_hi, w1_hi[...], preferred_element_type=jnp.float32)
        + b1[...], 0.0)
    lg = jnp.dot(s, w2[...], preferred_element_type=jnp.float32) + b2[...]
    shot[...] = lg[:B, :]


def _vspec(shape):
    return pl.BlockSpec(shape, lambda *args: tuple(0 for _ in shape))


def kernel(x, edge_index, edge_attr, receiver_mask, batch,
           W_msg1, b_msg1, W_upd1, b_upd1, W_msg2, b_msg2, W_upd2, b_upd2,
           W_recv, b_recv, W_shot1, b_shot1, W_shot2, b_shot2):
    f32 = jnp.float32
    src = edge_index[0]
    dst = edge_index[1]
    dst2d = dst.reshape(E // GS, GS)
    maskf = receiver_mask.astype(f32).reshape(N, 1)
    batch2 = batch.reshape(N, 1)

    wm1x_lo, wm1x_hi = W_msg1[:13, :HH], W_msg1[:13, HH:]
    _b16 = lambda w: w.astype(jnp.bfloat16).astype(jnp.float32)
    wm1i_lo, wm1i_hi = _b16(W_msg1[13:14, :HH]), _b16(W_msg1[13:14, HH:])
    wm1a_lo, wm1a_hi = W_msg1[14:18, :HH], W_msg1[14:18, HH:]
    bm1_lo, bm1_hi = b_msg1.reshape(1, H)[:, :HH], b_msg1.reshape(1, H)[:, HH:]
    wu1x_lo, wu1x_hi = W_upd1[:13, :HH], W_upd1[:13, HH:]
    wu1i_lo, wu1i_hi = _b16(W_upd1[13:14, :HH]), _b16(W_upd1[13:14, HH:])
    wu1a = W_upd1[14:78]
    wu1a_ll, wu1a_lh = wu1a[:HH, :HH], wu1a[:HH, HH:]
    wu1a_hl, wu1a_hh = wu1a[HH:, :HH], wu1a[HH:, HH:]
    bu1_lo, bu1_hi = b_upd1.reshape(1, H)[:, :HH], b_upd1.reshape(1, H)[:, HH:]
    wm2x = W_msg2[:64]
    wm2a_lo, wm2a_hi = W_msg2[64:68, :HH], W_msg2[64:68, HH:]
    bm2 = b_msg2.reshape(1, H)
    wu2h = W_upd2[:64]
    wu2a = W_upd2[64:128]
    bu2 = b_upd2.reshape(1, H)
    wu2a_lo = jnp.stack([wu2a[:HH, :HH], wu2a[:HH, HH:]])
    wu2a_hi = jnp.stack([wu2a[HH:, :HH], wu2a[HH:, HH:]])
    wr_lo = W_recv[:HH]
    wr_hi = W_recv[HH:]
    br = b_recv.reshape(1, 1)
    ws1_lo, ws1_hi = W_shot1[:HH], W_shot1[HH:]
    bs1 = b_shot1.reshape(1, H)
    ws2 = W_shot2
    bs2 = b_shot2.reshape(1, 1)

    cost_big = pl.CostEstimate(flops=2 * N * 78 * H, bytes_accessed=N * 600,
                               transcendentals=0)

    wspec13 = _vspec((13, HH))
    bspec = _vspec((1, HH))
    p1_st, xu_st = pl.pallas_call(
        _pre_k,
        grid=(2, NBN),
        in_specs=[pl.BlockSpec((BN, 13), lambda c, g: (g, 0))] +
                 [wspec13, wspec13, bspec, bspec, wspec13, wspec13, bspec,
                  bspec],
        out_specs=[pl.BlockSpec((BN, HH), lambda c, g: (c * NBN + g, 0))] * 2,
        out_shape=[jax.ShapeDtypeStruct((2 * N, HH), f32)] * 2,
    )(x, wm1x_lo, wm1x_hi, bm1_lo, bm1_hi, wu1x_lo, wu1x_hi, bu1_lo, bu1_hi)

    import jax.scipy.linalg as jsl
    attr_pk = edge_attr.reshape(E // 4, 16)
    _bd = lambda w: jsl.block_diag(w, w, w, w)
    wspec4 = _vspec((16, 4 * HH))
    BQ = 400
    NBQ = (E // 4) // BQ
    q1_st, q2_st = pl.pallas_call(
        _qtab_k,
        grid=(2, NBQ),
        in_specs=[pl.BlockSpec((BQ, 16), lambda c, g: (g, 0)),
                  wspec4, wspec4, wspec4, wspec4],
        out_specs=[pl.BlockSpec((BQ, 4 * HH),
                                lambda c, g: (c * NBQ + g, 0))] * 2,
        out_shape=[jax.ShapeDtypeStruct((2 * (E // 4), 4 * HH), f32)] * 2,
    )(attr_pk, _bd(wm1a_lo), _bd(wm1a_hi), _bd(wm2a_lo), _bd(wm2a_hi))

    wspecH = _vspec((HH, HH))

    def upd(base_st, agg_st, wll, wlh, whl, whh):
        return pl.pallas_call(
            _upd_k,
            grid=(2, NBN),
            in_specs=[pl.BlockSpec((BN, HH), lambda c, g: (c * NBN + g, 0)),
                      pl.BlockSpec((BN, HH), lambda c, g: (g, 0)),
                      pl.BlockSpec((BN, HH), lambda c, g: (NBN + g, 0)),
                      wspecH, wspecH, wspecH, wspecH],
            out_specs=pl.BlockSpec((BN, HH), lambda c, g: (c * NBN + g, 0)),
            out_shape=jax.ShapeDtypeStruct((2 * N, HH), f32),
            cost_estimate=cost_big,
        )(base_st, agg_st, agg_st, wll, wlh, whl, whh)

    def tab2(h_st, WA, bA, WB, bB):
        return pl.pallas_call(
            _tab2_k,
            grid=(2, NBN),
            in_specs=[pl.BlockSpec((BN, HH), lambda c, g: (g, 0)),
                      pl.BlockSpec((BN, HH), lambda c, g: (NBN + g, 0)),
                      wspecH, wspecH, wspecH, wspecH, bspec, bspec,
                      wspecH, wspecH, wspecH, wspecH, bspec, bspec],
            out_specs=[pl.BlockSpec((BN, HH),
                                    lambda c, g: (c * NBN + g, 0))] * 2,
            out_shape=[jax.ShapeDtypeStruct((2 * N, HH), f32)] * 2,
            cost_estimate=cost_big,
        )(h_st, h_st,
          WA[:HH, :HH], WA[:HH, HH:], WA[HH:, :HH], WA[HH:, HH:],
          bA[:, :HH], bA[:, HH:],
          WB[:HH, :HH], WB[:HH, HH:], WB[HH:, :HH], WB[HH:, HH:],
          bB[:, :HH], bB[:, HH:])

    def backbone_tail(p1t, xut):
        msg = _get_msg_call()
        agg1_st = msg(p1t, q1_st, src, dst2d)
        h1_st = upd(xut, agg1_st, wu1a_ll, wu1a_lh, wu1a_hl, wu1a_hh)
        p2_st, hu_st = tab2(h1_st, wm2x, bm2, wu2h, bu2)
        agg2_st = msg(p2_st, q2_st, src, dst2d)
        return hu_st, agg2_st

    hu_st, agg2_st = backbone_tail(p1_st, xu_st)
    logits = pl.pallas_call(
        _logits_k,
        grid=(NBN,),
        in_specs=[pl.BlockSpec((2, BN, HH), lambda g: (0, g, 0)),
                  pl.BlockSpec((BN, HH), lambda g: (g, 0)),
                  pl.BlockSpec((BN, HH), lambda g: (NBN + g, 0)),
                  _vspec((2, HH, HH)), _vspec((2, HH, HH)),
                  _vspec((HH, 1)), _vspec((HH, 1)), _vspec((1, 1))],
        out_specs=pl.BlockSpec((BN, 1), lambda g: (g, 0)),
        out_shape=jax.ShapeDtypeStruct((N, 1), f32),
        cost_estimate=cost_big,
    )(hu_st.reshape(2, N, HH), agg2_st, agg2_st, wu2a_lo, wu2a_hi,
      wr_lo, wr_hi, br)

    nspec = pl.BlockSpec((BN, 1), lambda g: (g, 0))
    bl_spec = _vspec((1, BL))
    segargs = dict(grid=(NBN,))
    m_seg = pl.pallas_call(
        _segmax_k, in_specs=[nspec, nspec, nspec], out_specs=bl_spec,
        out_shape=jax.ShapeDtypeStruct((1, BL), f32),
        scratch_shapes=[pltpu.VMEM((1, BL), f32)], **segargs,
    )(logits, maskf, batch2)
    e_arr, denom = pl.pallas_call(
        _exp_k, in_specs=[nspec, nspec, nspec, bl_spec],
        out_specs=[nspec, bl_spec],
        out_shape=[jax.ShapeDtypeStruct((N, 1), f32),
                   jax.ShapeDtypeStruct((1, BL), f32)],
        scratch_shapes=[pltpu.VMEM((1, BL), f32)], **segargs,
    )(logits, maskf, batch2, m_seg)
    probs, pm, ss = pl.pallas_call(
        _probs_k, in_specs=[nspec, nspec, bl_spec],
        out_specs=[nspec, bl_spec, bl_spec],
        out_shape=[jax.ShapeDtypeStruct((N, 1), f32),
                   jax.ShapeDtypeStruct((1, BL), f32),
                   jax.ShapeDtypeStruct((1, BL), f32)],
        scratch_shapes=[pltpu.VMEM((1, BL), f32)] * 2, **segargs,
    )(e_arr, batch2, denom)
    tgt = pl.pallas_call(
        _first_k, in_specs=[nspec, nspec, bl_spec, bl_spec],
        out_specs=pl.BlockSpec((1, BL), lambda g: (0, 0)),
        out_shape=jax.ShapeDtypeStruct((1, BL), jnp.int32),
        scratch_shapes=[pltpu.VMEM((1, BL), jnp.int32)], **segargs,
    )(probs, batch2, pm, ss)

    wspec1 = _vspec((1, HH))
    ind, p1p_st, xup_st = pl.pallas_call(
        _fix_k,
        grid=(2, NBN),
        in_specs=[pl.BlockSpec((BN, 1), lambda c, g: (g, 0)),
                  pl.BlockSpec((1, BL), lambda c, g: (0, 0)),
                  pl.BlockSpec((BN, HH), lambda c, g: (c * NBN + g, 0)),
                  pl.BlockSpec((BN, HH), lambda c, g: (c * NBN + g, 0)),
                  wspec1, wspec1, wspec1, wspec1],
        out_specs=[pl.BlockSpec((BN, 1), lambda c, g: (g, 0)),
                   pl.BlockSpec((BN, HH), lambda c, g: (c * NBN + g, 0)),
                   pl.BlockSpec((BN, HH), lambda c, g: (c * NBN + g, 0))],
        out_shape=[jax.ShapeDtypeStruct((N, 1), f32),
                   jax.ShapeDtypeStruct((2 * N, HH), f32),
                   jax.ShapeDtypeStruct((2 * N, HH), f32)],
    )(batch2, tgt, p1_st, xu_st, wm1i_lo, wm1i_hi, wu1i_lo, wu1i_hi)

    hu2_st, agg2b_st = backbone_tail(p1p_st, xup_st)
    pooled, counts = pl.pallas_call(
        _pool_k,
        grid=(2, NBN),
        in_specs=[pl.BlockSpec((BN, HH), lambda c, g: (c * NBN + g, 0)),
                  pl.BlockSpec((BN, HH), lambda c, g: (g, 0)),
                  pl.BlockSpec((BN, HH), lambda c, g: (NBN + g, 0)),
                  _vspec((2, HH, HH)), _vspec((2, HH, HH)),
                  pl.BlockSpec((BN, 1), lambda c, g: (g, 0))],
        out_specs=[pl.BlockSpec((1, BL, HH), lambda c, g: (c, 0, 0)),
                   pl.BlockSpec((1, BL), lambda c, g: (0, 0))],
        out_shape=[jax.ShapeDtypeStruct((2, BL, HH), f32),
                   jax.ShapeDtypeStruct((1, BL), f32)],
        scratch_shapes=[pltpu.VMEM((BL, HH), f32), pltpu.VMEM((1, BL), f32)],
        cost_estimate=cost_big,
    )(hu2_st, agg2b_st, agg2b_st, wu2a_lo, wu2a_hi, batch2)

    shot = pl.pallas_call(
        _shot_k,
        in_specs=[_vspec((2, BL, HH)), _vspec((1, BL)),
                  _vspec((HH, H)), _vspec((HH, H)), _vspec((1, H)),
                  _vspec((H, 1)), _vspec((1, 1))],
        out_specs=_vspec((B, 1)),
        out_shape=jax.ShapeDtypeStruct((B, 1), f32),
    )(pooled, counts, ws1_lo, ws1_hi, bs1, ws2, bs2)

    return probs[:, 0], ind[:, 0], shot
